# Initial kernel scaffold; baseline (speedup 1.0000x reference)
import jax
import jax.numpy as jnp
from jax.experimental import pallas as pl


def kernel(x, edge_index, edge_attr, en1_w1, en1_b1, en1_w2, en1_b2, root1, bias1, en2_w1, en2_b1, en2_w2, en2_b2, root2, bias2, lin_w, lin_b):
    raise NotImplementedError("write your pallas kernel here")


# speedup vs baseline: 1.2350x; 1.2350x over previous
import jax
import jax.numpy as jnp
from jax import lax
from jax.experimental import pallas as pl
from jax.experimental.pallas import tpu as pltpu
from jax.experimental.pallas import tpu_sc as plsc

N = 10000
E = 320000
C = 16
H = 256
NCLS = 64
W = 128
R = E // W
NW = 32
KF = R // NW
TAIL = R - KF * NW
NPC = N // 16


def _sc_gather_body(x_hbm, src_hbm, out_hbm, idx_v, rows_v, sem):
    cid = lax.axis_index("c")
    sid = lax.axis_index("s")
    wid = sid * 2 + cid

    def do_row(row):
        pltpu.sync_copy(src_hbm.at[row], idx_v)
        pltpu.async_copy(x_hbm.at[idx_v], rows_v, sem).wait()
        pltpu.sync_copy(rows_v, out_hbm.at[pl.ds(row * W, W)])

    def body(k, carry):
        do_row(k * NW + wid)
        return carry

    lax.fori_loop(0, KF, body, 0)

    @pl.when(wid < TAIL)
    def _():
        do_row(KF * NW + wid)


def _scatter_common(msg_hbm, dst_hbm, zeros_hbm, agg_out, idx_v, rows_v,
                    stage_v, agg_s, extra_row=None, extra_out=None,
                    ones_v=None, cnt_s=None):
    cid = lax.axis_index("c")
    sid = lax.axis_index("s")
    wid = sid * 2 + cid

    pltpu.sync_copy(zeros_hbm, stage_v)
    pltpu.sync_copy(stage_v, agg_s.at[pl.ds(sid * NPC, NPC)])
    if cnt_s is not None:
        pltpu.sync_copy(stage_v, cnt_s.at[pl.ds(sid * NPC, NPC)])
        pltpu.sync_copy(extra_row, ones_v)
    plsc.subcore_barrier()

    def do_row(row):
        pltpu.sync_copy(dst_hbm.at[row], idx_v)
        pltpu.sync_copy(msg_hbm.at[pl.ds(row * W, W)], rows_v)
        pltpu.sync_copy(rows_v, agg_s.at[idx_v], add=True)
        if cnt_s is not None:
            pltpu.sync_copy(ones_v, cnt_s.at[idx_v], add=True)

    def body(k, carry):
        do_row(k * NW + wid)
        return carry

    lax.fori_loop(0, KF, body, 0)

    @pl.when(wid < TAIL)
    def _():
        do_row(KF * NW + wid)

    plsc.subcore_barrier()

    pltpu.sync_copy(agg_s.at[pl.ds(sid * NPC, NPC)], stage_v)
    pltpu.sync_copy(stage_v, agg_out.at[cid, pl.ds(sid * NPC, NPC)])
    if cnt_s is not None:
        pltpu.sync_copy(cnt_s.at[pl.ds(sid * NPC, NPC)], stage_v)
        pltpu.sync_copy(stage_v, extra_out.at[cid, pl.ds(sid * NPC, NPC)])


def _sc_scatter_cnt_body(msg_hbm, dst_hbm, zeros_hbm, ones_hbm, agg_out,
                         cnt_out, idx_v, rows_v, stage_v, ones_v, agg_s,
                         cnt_s):
    _scatter_common(msg_hbm, dst_hbm, zeros_hbm, agg_out, idx_v, rows_v,
                    stage_v, agg_s, extra_row=ones_hbm, extra_out=cnt_out,
                    ones_v=ones_v, cnt_s=cnt_s)


def _sc_scatter_body(msg_hbm, dst_hbm, zeros_hbm, agg_out, idx_v, rows_v,
                     stage_v, agg_s):
    _scatter_common(msg_hbm, dst_hbm, zeros_hbm, agg_out, idx_v, rows_v,
                    stage_v, agg_s)


import functools


@functools.lru_cache(maxsize=1)
def _sc_kernels():
    mesh = plsc.VectorSubcoreMesh(core_axis_name="c", subcore_axis_name="s")
    params = pltpu.CompilerParams(use_tc_tiling_on_sc=False)
    gather = pl.kernel(
        _sc_gather_body,
        out_type=jax.ShapeDtypeStruct((E, C), jnp.float32),
        mesh=mesh,
        compiler_params=params,
        scratch_types=[
            pltpu.VMEM((W,), jnp.int32),
            pltpu.VMEM((W, C), jnp.float32),
            pltpu.SemaphoreType.DMA,
        ],
    )
    scatter_cnt = pl.kernel(
        _sc_scatter_cnt_body,
        compiler_params=params,
        out_type=(
            jax.ShapeDtypeStruct((2, N, C), jnp.float32),
            jax.ShapeDtypeStruct((2, N, C), jnp.float32),
        ),
        mesh=mesh,
        scratch_types=[
            pltpu.VMEM((W,), jnp.int32),
            pltpu.VMEM((W, C), jnp.float32),
            pltpu.VMEM((NPC, C), jnp.float32),
            pltpu.VMEM((W, C), jnp.float32),
            pltpu.VMEM_SHARED((N, C), jnp.float32),
            pltpu.VMEM_SHARED((N, C), jnp.float32),
        ],
    )
    scatter = pl.kernel(
        _sc_scatter_body,
        compiler_params=params,
        out_type=jax.ShapeDtypeStruct((2, N, C), jnp.float32),
        mesh=mesh,
        scratch_types=[
            pltpu.VMEM((W,), jnp.int32),
            pltpu.VMEM((W, C), jnp.float32),
            pltpu.VMEM((NPC, C), jnp.float32),
            pltpu.VMEM_SHARED((N, C), jnp.float32),
        ],
    )
    return gather, scatter_cnt, scatter


BE = 512
GE = E // BE


def _tc_msg_body(ea_ref, xs_ref, w1_ref, b1_ref, w2_ref, b2_ref, out_ref):
    t = jnp.maximum(
        jnp.dot(ea_ref[...], w1_ref[...], preferred_element_type=jnp.float32)
        + b1_ref[...], 0.0)
    h = jnp.dot(t, w2_ref[...], preferred_element_type=jnp.float32) + b2_ref[...]
    xs = xs_ref[...]
    msg = xs[:, 0:1] * h[:, 0:C]
    for i in range(1, C):
        msg = msg + xs[:, i:i + 1] * h[:, i * C:(i + 1) * C]
    out_ref[...] = msg


_tc_msg = pl.pallas_call(
    _tc_msg_body,
    grid=(GE,),
    in_specs=[
        pl.BlockSpec((BE, C), lambda i: (i, 0)),
        pl.BlockSpec((BE, C), lambda i: (i, 0)),
        pl.BlockSpec((C, H), lambda i: (0, 0)),
        pl.BlockSpec((1, H), lambda i: (0, 0)),
        pl.BlockSpec((H, H), lambda i: (0, 0)),
        pl.BlockSpec((1, H), lambda i: (0, 0)),
    ],
    out_specs=pl.BlockSpec((BE, C), lambda i: (i, 0)),
    out_shape=jax.ShapeDtypeStruct((E, C), jnp.float32),
)


def _tc_combine1_body(x_ref, agg_ref, cnt_ref, root_ref, bias_ref, out_ref):
    aggv = agg_ref[0] + agg_ref[1]
    cntv = jnp.maximum(cnt_ref[0] + cnt_ref[1], 1.0)
    mean = aggv / cntv
    out_ref[...] = jnp.maximum(
        jnp.dot(x_ref[...], root_ref[...], preferred_element_type=jnp.float32)
        + mean + bias_ref[...], 0.0)


_tc_combine1 = pl.pallas_call(
    _tc_combine1_body,
    out_shape=jax.ShapeDtypeStruct((N, C), jnp.float32),
)


def _tc_combine2_body(h_ref, agg_ref, cnt_ref, root_ref, bias_ref, lw_ref,
                      lb_ref, out_ref):
    aggv = agg_ref[0] + agg_ref[1]
    cntv = jnp.maximum(cnt_ref[0] + cnt_ref[1], 1.0)
    h2 = jnp.maximum(
        jnp.dot(h_ref[...], root_ref[...], preferred_element_type=jnp.float32)
        + aggv / cntv + bias_ref[...], 0.0)
    out_ref[...] = jnp.dot(
        h2, lw_ref[...], preferred_element_type=jnp.float32) + lb_ref[...]


_tc_combine2 = pl.pallas_call(
    _tc_combine2_body,
    out_shape=jax.ShapeDtypeStruct((N, NCLS), jnp.float32),
)


def kernel(x, edge_index, edge_attr,
           en1_w1, en1_b1, en1_w2, en1_b2, root1, bias1,
           en2_w1, en2_b1, en2_w2, en2_b2, root2, bias2,
           lin_w, lin_b):
    src2d = edge_index[0].reshape(R, W)
    dst2d = edge_index[1].reshape(R, W)
    zeros = jnp.zeros((NPC, C), jnp.float32)
    ones = jnp.ones((W, C), jnp.float32)
    _sc_gather, _sc_scatter_cnt, _sc_scatter = _sc_kernels()

    xs = _sc_gather(x, src2d)
    msg1 = _tc_msg(edge_attr, xs, en1_w1, en1_b1.reshape(1, H),
                   en1_w2, en1_b2.reshape(1, H))
    agg1, cnt = _sc_scatter_cnt(msg1, dst2d, zeros, ones)
    h1 = _tc_combine1(x, agg1, cnt, root1, bias1.reshape(1, C))

    hs = _sc_gather(h1, src2d)
    msg2 = _tc_msg(edge_attr, hs, en2_w1, en2_b1.reshape(1, H),
                   en2_w2, en2_b2.reshape(1, H))
    agg2 = _sc_scatter(msg2, dst2d, zeros)
    return _tc_combine2(h1, agg2, cnt, root2, bias2.reshape(1, C),
                        lin_w, lin_b.reshape(1, NCLS))


# speedup vs baseline: 2.3349x; 1.8907x over previous
import jax
import jax.numpy as jnp
from jax import lax
from jax.experimental import pallas as pl
from jax.experimental.pallas import tpu as pltpu
from jax.experimental.pallas import tpu_sc as plsc

N = 10000
E = 320000
C = 16
H = 256
NCLS = 64
W = 128
R = E // W
NW = 32
KF = R // NW
TAIL = R - KF * NW
NPC = N // 16


def _sc_gather_body(x_hbm, src_hbm, out_hbm, idx_v, rows_v, sem):
    cid = lax.axis_index("c")
    sid = lax.axis_index("s")
    wid = sid * 2 + cid

    def do_row(row):
        pltpu.sync_copy(src_hbm.at[row], idx_v)
        pltpu.async_copy(x_hbm.at[idx_v], rows_v, sem).wait()
        pltpu.sync_copy(rows_v, out_hbm.at[pl.ds(row * W, W)])

    def body(k, carry):
        do_row(k * NW + wid)
        return carry

    lax.fori_loop(0, KF, body, 0)

    @pl.when(wid < TAIL)
    def _():
        do_row(KF * NW + wid)


def _scatter_common(msg_hbm, dst_hbm, zeros_hbm, agg_out, idx_v, rows_v,
                    stage_v, agg_s, extra_row=None, extra_out=None,
                    ones_v=None, cnt_s=None):
    cid = lax.axis_index("c")
    sid = lax.axis_index("s")
    wid = sid * 2 + cid

    pltpu.sync_copy(zeros_hbm, stage_v)
    pltpu.sync_copy(stage_v, agg_s.at[pl.ds(sid * NPC, NPC)])
    if cnt_s is not None:
        pltpu.sync_copy(stage_v, cnt_s.at[pl.ds(sid * NPC, NPC)])
        pltpu.sync_copy(extra_row, ones_v)
    plsc.subcore_barrier()

    def do_row(row):
        pltpu.sync_copy(dst_hbm.at[row], idx_v)
        pltpu.sync_copy(msg_hbm.at[pl.ds(row * W, W)], rows_v)
        pltpu.sync_copy(rows_v, agg_s.at[idx_v], add=True)
        if cnt_s is not None:
            pltpu.sync_copy(ones_v, cnt_s.at[idx_v], add=True)

    def body(k, carry):
        do_row(k * NW + wid)
        return carry

    lax.fori_loop(0, KF, body, 0)

    @pl.when(wid < TAIL)
    def _():
        do_row(KF * NW + wid)

    plsc.subcore_barrier()

    pltpu.sync_copy(agg_s.at[pl.ds(sid * NPC, NPC)], stage_v)
    pltpu.sync_copy(stage_v, agg_out.at[cid, pl.ds(sid * NPC, NPC)])
    if cnt_s is not None:
        pltpu.sync_copy(cnt_s.at[pl.ds(sid * NPC, NPC)], stage_v)
        pltpu.sync_copy(stage_v, extra_out.at[cid, pl.ds(sid * NPC, NPC)])


def _sc_scatter_cnt_body(msg_hbm, dst_hbm, zeros_hbm, ones_hbm, agg_out,
                         cnt_out, idx_v, rows_v, stage_v, ones_v, agg_s,
                         cnt_s):
    _scatter_common(msg_hbm, dst_hbm, zeros_hbm, agg_out, idx_v, rows_v,
                    stage_v, agg_s, extra_row=ones_hbm, extra_out=cnt_out,
                    ones_v=ones_v, cnt_s=cnt_s)


def _sc_scatter_body(msg_hbm, dst_hbm, zeros_hbm, agg_out, idx_v, rows_v,
                     stage_v, agg_s):
    _scatter_common(msg_hbm, dst_hbm, zeros_hbm, agg_out, idx_v, rows_v,
                    stage_v, agg_s)


import functools


@functools.lru_cache(maxsize=1)
def _sc_kernels():
    mesh = plsc.VectorSubcoreMesh(core_axis_name="c", subcore_axis_name="s")
    params = pltpu.CompilerParams(use_tc_tiling_on_sc=False)
    gather = pl.kernel(
        _sc_gather_body,
        out_type=jax.ShapeDtypeStruct((E, C), jnp.float32),
        mesh=mesh,
        compiler_params=params,
        scratch_types=[
            pltpu.VMEM((W,), jnp.int32),
            pltpu.VMEM((W, C), jnp.float32),
            pltpu.SemaphoreType.DMA,
        ],
    )
    scatter_cnt = pl.kernel(
        _sc_scatter_cnt_body,
        compiler_params=params,
        out_type=(
            jax.ShapeDtypeStruct((2, N, C), jnp.float32),
            jax.ShapeDtypeStruct((2, N, C), jnp.float32),
        ),
        mesh=mesh,
        scratch_types=[
            pltpu.VMEM((W,), jnp.int32),
            pltpu.VMEM((W, C), jnp.float32),
            pltpu.VMEM((NPC, C), jnp.float32),
            pltpu.VMEM((W, C), jnp.float32),
            pltpu.VMEM_SHARED((N, C), jnp.float32),
            pltpu.VMEM_SHARED((N, C), jnp.float32),
        ],
    )
    scatter = pl.kernel(
        _sc_scatter_body,
        compiler_params=params,
        out_type=jax.ShapeDtypeStruct((2, N, C), jnp.float32),
        mesh=mesh,
        scratch_types=[
            pltpu.VMEM((W,), jnp.int32),
            pltpu.VMEM((W, C), jnp.float32),
            pltpu.VMEM((NPC, C), jnp.float32),
            pltpu.VMEM_SHARED((N, C), jnp.float32),
        ],
    )
    return gather, scatter_cnt, scatter


BE = 512
GE = E // BE


def _tc_msg_body(ea_ref, xs_ref, w1_ref, b1_ref, w2_ref, b2_ref, exp_ref,
                 red_ref, out_ref):
    t = jnp.maximum(
        jnp.dot(ea_ref[...], w1_ref[...], preferred_element_type=jnp.float32)
        + b1_ref[...], 0.0)
    h = jnp.dot(t, w2_ref[...], preferred_element_type=jnp.float32) + b2_ref[...]
    xs_rep = jnp.dot(xs_ref[...], exp_ref[...],
                     preferred_element_type=jnp.float32)
    out_ref[...] = jnp.dot(xs_rep * h, red_ref[...],
                           preferred_element_type=jnp.float32)


_tc_msg = pl.pallas_call(
    _tc_msg_body,
    grid=(GE,),
    in_specs=[
        pl.BlockSpec((BE, C), lambda i: (i, 0)),
        pl.BlockSpec((BE, C), lambda i: (i, 0)),
        pl.BlockSpec((C, H), lambda i: (0, 0)),
        pl.BlockSpec((1, H), lambda i: (0, 0)),
        pl.BlockSpec((H, H), lambda i: (0, 0)),
        pl.BlockSpec((1, H), lambda i: (0, 0)),
        pl.BlockSpec((C, H), lambda i: (0, 0)),
        pl.BlockSpec((H, C), lambda i: (0, 0)),
    ],
    out_specs=pl.BlockSpec((BE, C), lambda i: (i, 0)),
    out_shape=jax.ShapeDtypeStruct((E, C), jnp.float32),
)


def _tc_combine1_body(x_ref, agg_ref, cnt_ref, root_ref, bias_ref, out_ref):
    aggv = agg_ref[0] + agg_ref[1]
    cntv = jnp.maximum(cnt_ref[0] + cnt_ref[1], 1.0)
    mean = aggv / cntv
    out_ref[...] = jnp.maximum(
        jnp.dot(x_ref[...], root_ref[...], preferred_element_type=jnp.float32)
        + mean + bias_ref[...], 0.0)


_tc_combine1 = pl.pallas_call(
    _tc_combine1_body,
    out_shape=jax.ShapeDtypeStruct((N, C), jnp.float32),
)


def _tc_combine2_body(h_ref, agg_ref, cnt_ref, root_ref, bias_ref, lw_ref,
                      lb_ref, out_ref):
    aggv = agg_ref[0] + agg_ref[1]
    cntv = jnp.maximum(cnt_ref[0] + cnt_ref[1], 1.0)
    h2 = jnp.maximum(
        jnp.dot(h_ref[...], root_ref[...], preferred_element_type=jnp.float32)
        + aggv / cntv + bias_ref[...], 0.0)
    out_ref[...] = jnp.dot(
        h2, lw_ref[...], preferred_element_type=jnp.float32) + lb_ref[...]


_tc_combine2 = pl.pallas_call(
    _tc_combine2_body,
    out_shape=jax.ShapeDtypeStruct((N, NCLS), jnp.float32),
)


def kernel(x, edge_index, edge_attr,
           en1_w1, en1_b1, en1_w2, en1_b2, root1, bias1,
           en2_w1, en2_b1, en2_w2, en2_b2, root2, bias2,
           lin_w, lin_b):
    src2d = edge_index[0].reshape(R, W)
    dst2d = edge_index[1].reshape(R, W)
    zeros = jnp.zeros((NPC, C), jnp.float32)
    ones = jnp.ones((W, C), jnp.float32)
    jidx = jnp.arange(H) // C
    exp_m = (jidx[None, :] == jnp.arange(C)[:, None]).astype(jnp.float32)
    red_m = jnp.tile(jnp.eye(C, dtype=jnp.float32), (C, 1))
    _sc_gather, _sc_scatter_cnt, _sc_scatter = _sc_kernels()

    xs = _sc_gather(x, src2d)
    msg1 = _tc_msg(edge_attr, xs, en1_w1, en1_b1.reshape(1, H),
                   en1_w2, en1_b2.reshape(1, H), exp_m, red_m)
    agg1, cnt = _sc_scatter_cnt(msg1, dst2d, zeros, ones)
    h1 = _tc_combine1(x, agg1, cnt, root1, bias1.reshape(1, C))

    hs = _sc_gather(h1, src2d)
    msg2 = _tc_msg(edge_attr, hs, en2_w1, en2_b1.reshape(1, H),
                   en2_w2, en2_b2.reshape(1, H), exp_m, red_m)
    agg2 = _sc_scatter(msg2, dst2d, zeros)
    return _tc_combine2(h1, agg2, cnt, root2, bias2.reshape(1, C),
                        lin_w, lin_b.reshape(1, NCLS))


# speedup vs baseline: 3.9467x; 1.6903x over previous
import jax
import jax.numpy as jnp
from jax import lax
from jax.experimental import pallas as pl
from jax.experimental.pallas import tpu as pltpu
from jax.experimental.pallas import tpu_sc as plsc

N = 10000
E = 320000
C = 16
H = 256
NCLS = 64
W = 128
R = E // W
NW = 32
KF = R // NW
TAIL = R - KF * NW
KB = 13
NB = KF // KB
NPC = N // 16


def _sc_gather_body(x_hbm, src_hbm, out_hbm, idx_v, rows_v, sem):
    cid = lax.axis_index("c")
    sid = lax.axis_index("s")
    wid = sid * 2 + cid
    start = wid * KF

    def do_batch(b, carry):
        row0 = start + b * KB
        pltpu.sync_copy(src_hbm.at[pl.ds(row0, KB)], idx_v)
        waits = []
        for j in range(KB):
            waits.append(pltpu.async_copy(
                x_hbm.at[idx_v.at[j]], rows_v.at[pl.ds(j * W, W)], sem))
        for wdesc in waits:
            wdesc.wait()
        pltpu.sync_copy(rows_v, out_hbm.at[pl.ds(row0 * W, KB * W)])
        return carry

    lax.fori_loop(0, NB, do_batch, 0)

    @pl.when(wid < TAIL)
    def _():
        row = NW * KF + wid
        pltpu.sync_copy(src_hbm.at[row], idx_v.at[0])
        pltpu.async_copy(x_hbm.at[idx_v.at[0]], rows_v.at[pl.ds(0, W)],
                         sem).wait()
        pltpu.sync_copy(rows_v.at[pl.ds(0, W)], out_hbm.at[pl.ds(row * W, W)])


def _scatter_common(msg_hbm, dst_hbm, zeros_hbm, agg_out, idx_v, rows_v,
                    stage_v, agg_s, sem, extra_row=None, extra_out=None,
                    ones_v=None, cnt_s=None):
    cid = lax.axis_index("c")
    sid = lax.axis_index("s")
    wid = sid * 2 + cid

    pltpu.sync_copy(zeros_hbm, stage_v)
    pltpu.sync_copy(stage_v, agg_s.at[pl.ds(sid * NPC, NPC)])
    if cnt_s is not None:
        pltpu.sync_copy(stage_v, cnt_s.at[pl.ds(sid * NPC, NPC)])
        pltpu.sync_copy(extra_row, ones_v)
    plsc.subcore_barrier()

    start = wid * KF

    def do_batch(b, carry):
        row0 = start + b * KB
        pltpu.sync_copy(dst_hbm.at[pl.ds(row0, KB)], idx_v)
        pltpu.sync_copy(msg_hbm.at[pl.ds(row0 * W, KB * W)], rows_v)
        waits = []
        for j in range(KB):
            waits.append(pltpu.async_copy(
                rows_v.at[pl.ds(j * W, W)], agg_s.at[idx_v.at[j]], sem,
                add=True))
            if cnt_s is not None:
                waits.append(pltpu.async_copy(
                    ones_v, cnt_s.at[idx_v.at[j]], sem, add=True))
        for wdesc in waits:
            wdesc.wait()
        return carry

    lax.fori_loop(0, NB, do_batch, 0)

    @pl.when(wid < TAIL)
    def _():
        row = NW * KF + wid
        pltpu.sync_copy(dst_hbm.at[row], idx_v.at[0])
        pltpu.sync_copy(msg_hbm.at[pl.ds(row * W, W)], rows_v.at[pl.ds(0, W)])
        pltpu.sync_copy(rows_v.at[pl.ds(0, W)], agg_s.at[idx_v.at[0]],
                        add=True)
        if cnt_s is not None:
            pltpu.sync_copy(ones_v, cnt_s.at[idx_v.at[0]], add=True)

    plsc.subcore_barrier()

    pltpu.sync_copy(agg_s.at[pl.ds(sid * NPC, NPC)], stage_v)
    pltpu.sync_copy(stage_v, agg_out.at[cid, pl.ds(sid * NPC, NPC)])
    if cnt_s is not None:
        pltpu.sync_copy(cnt_s.at[pl.ds(sid * NPC, NPC)], stage_v)
        pltpu.sync_copy(stage_v, extra_out.at[cid, pl.ds(sid * NPC, NPC)])


def _sc_scatter_cnt_body(msg_hbm, dst_hbm, zeros_hbm, ones_hbm, agg_out,
                         cnt_out, idx_v, rows_v, stage_v, ones_v, agg_s,
                         cnt_s, sem):
    _scatter_common(msg_hbm, dst_hbm, zeros_hbm, agg_out, idx_v, rows_v,
                    stage_v, agg_s, sem, extra_row=ones_hbm,
                    extra_out=cnt_out, ones_v=ones_v, cnt_s=cnt_s)


def _sc_scatter_body(msg_hbm, dst_hbm, zeros_hbm, agg_out, idx_v, rows_v,
                     stage_v, agg_s, sem):
    _scatter_common(msg_hbm, dst_hbm, zeros_hbm, agg_out, idx_v, rows_v,
                    stage_v, agg_s, sem)


import functools


@functools.lru_cache(maxsize=1)
def _sc_kernels():
    mesh = plsc.VectorSubcoreMesh(core_axis_name="c", subcore_axis_name="s")
    params = pltpu.CompilerParams(use_tc_tiling_on_sc=False)
    gather = pl.kernel(
        _sc_gather_body,
        out_type=jax.ShapeDtypeStruct((E, C), jnp.float32),
        mesh=mesh,
        compiler_params=params,
        scratch_types=[
            pltpu.VMEM((KB, W), jnp.int32),
            pltpu.VMEM((KB * W, C), jnp.float32),
            pltpu.SemaphoreType.DMA,
        ],
    )
    scatter_cnt = pl.kernel(
        _sc_scatter_cnt_body,
        compiler_params=params,
        out_type=(
            jax.ShapeDtypeStruct((2, N, C), jnp.float32),
            jax.ShapeDtypeStruct((2, N, C), jnp.float32),
        ),
        mesh=mesh,
        scratch_types=[
            pltpu.VMEM((KB, W), jnp.int32),
            pltpu.VMEM((KB * W, C), jnp.float32),
            pltpu.VMEM((NPC, C), jnp.float32),
            pltpu.VMEM((W, C), jnp.float32),
            pltpu.VMEM_SHARED((N, C), jnp.float32),
            pltpu.VMEM_SHARED((N, C), jnp.float32),
            pltpu.SemaphoreType.DMA,
        ],
    )
    scatter = pl.kernel(
        _sc_scatter_body,
        compiler_params=params,
        out_type=jax.ShapeDtypeStruct((2, N, C), jnp.float32),
        mesh=mesh,
        scratch_types=[
            pltpu.VMEM((KB, W), jnp.int32),
            pltpu.VMEM((KB * W, C), jnp.float32),
            pltpu.VMEM((NPC, C), jnp.float32),
            pltpu.VMEM_SHARED((N, C), jnp.float32),
            pltpu.SemaphoreType.DMA,
        ],
    )
    return gather, scatter_cnt, scatter


BE = 2000
GE = E // BE


def _tc_msg_body(ea_ref, xs_ref, w1_ref, b1_ref, w2_ref, b2_ref, exp_ref,
                 red_ref, out_ref):
    t = jnp.maximum(
        jnp.dot(ea_ref[...], w1_ref[...], preferred_element_type=jnp.float32)
        + b1_ref[...], 0.0)
    h = jnp.dot(t, w2_ref[...], preferred_element_type=jnp.float32) + b2_ref[...]
    xs_rep = jnp.dot(xs_ref[...], exp_ref[...],
                     preferred_element_type=jnp.float32)
    out_ref[...] = jnp.dot(xs_rep * h, red_ref[...],
                           preferred_element_type=jnp.float32)


_tc_msg = pl.pallas_call(
    _tc_msg_body,
    grid=(GE,),
    in_specs=[
        pl.BlockSpec((BE, C), lambda i: (i, 0)),
        pl.BlockSpec((BE, C), lambda i: (i, 0)),
        pl.BlockSpec((C, H), lambda i: (0, 0)),
        pl.BlockSpec((1, H), lambda i: (0, 0)),
        pl.BlockSpec((H, H), lambda i: (0, 0)),
        pl.BlockSpec((1, H), lambda i: (0, 0)),
        pl.BlockSpec((C, H), lambda i: (0, 0)),
        pl.BlockSpec((H, C), lambda i: (0, 0)),
    ],
    out_specs=pl.BlockSpec((BE, C), lambda i: (i, 0)),
    out_shape=jax.ShapeDtypeStruct((E, C), jnp.float32),
)


def _tc_combine1_body(x_ref, agg_ref, cnt_ref, root_ref, bias_ref, out_ref):
    aggv = agg_ref[0] + agg_ref[1]
    cntv = jnp.maximum(cnt_ref[0] + cnt_ref[1], 1.0)
    mean = aggv / cntv
    out_ref[...] = jnp.maximum(
        jnp.dot(x_ref[...], root_ref[...], preferred_element_type=jnp.float32)
        + mean + bias_ref[...], 0.0)


_tc_combine1 = pl.pallas_call(
    _tc_combine1_body,
    out_shape=jax.ShapeDtypeStruct((N, C), jnp.float32),
)


def _tc_combine2_body(h_ref, agg_ref, cnt_ref, root_ref, bias_ref, lw_ref,
                      lb_ref, out_ref):
    aggv = agg_ref[0] + agg_ref[1]
    cntv = jnp.maximum(cnt_ref[0] + cnt_ref[1], 1.0)
    h2 = jnp.maximum(
        jnp.dot(h_ref[...], root_ref[...], preferred_element_type=jnp.float32)
        + aggv / cntv + bias_ref[...], 0.0)
    out_ref[...] = jnp.dot(
        h2, lw_ref[...], preferred_element_type=jnp.float32) + lb_ref[...]


_tc_combine2 = pl.pallas_call(
    _tc_combine2_body,
    out_shape=jax.ShapeDtypeStruct((N, NCLS), jnp.float32),
)


def kernel(x, edge_index, edge_attr,
           en1_w1, en1_b1, en1_w2, en1_b2, root1, bias1,
           en2_w1, en2_b1, en2_w2, en2_b2, root2, bias2,
           lin_w, lin_b):
    src2d = edge_index[0].reshape(R, W)
    dst2d = edge_index[1].reshape(R, W)
    zeros = jnp.zeros((NPC, C), jnp.float32)
    ones = jnp.ones((W, C), jnp.float32)
    jidx = jnp.arange(H) // C
    exp_m = (jidx[None, :] == jnp.arange(C)[:, None]).astype(jnp.float32)
    red_m = jnp.tile(jnp.eye(C, dtype=jnp.float32), (C, 1))
    _sc_gather, _sc_scatter_cnt, _sc_scatter = _sc_kernels()

    xs = _sc_gather(x, src2d)
    msg1 = _tc_msg(edge_attr, xs, en1_w1, en1_b1.reshape(1, H),
                   en1_w2, en1_b2.reshape(1, H), exp_m, red_m)
    agg1, cnt = _sc_scatter_cnt(msg1, dst2d, zeros, ones)
    h1 = _tc_combine1(x, agg1, cnt, root1, bias1.reshape(1, C))

    hs = _sc_gather(h1, src2d)
    msg2 = _tc_msg(edge_attr, hs, en2_w1, en2_b1.reshape(1, H),
                   en2_w2, en2_b2.reshape(1, H), exp_m, red_m)
    agg2 = _sc_scatter(msg2, dst2d, zeros)
    return _tc_combine2(h1, agg2, cnt, root2, bias2.reshape(1, C),
                        lin_w, lin_b.reshape(1, NCLS))


# speedup vs baseline: 4.0917x; 1.0367x over previous
import jax
import jax.numpy as jnp
from jax import lax
from jax.experimental import pallas as pl
from jax.experimental.pallas import tpu as pltpu
from jax.experimental.pallas import tpu_sc as plsc

N = 10000
E = 320000
C = 16
H = 256
NCLS = 64
W = 128
R = E // W
NW = 32
KF = R // NW
TAIL = R - KF * NW
KB = 13
NB = KF // KB
SKB = 8
NBT = R // 8
TAILR = R - NBT * 8
NBATCH = (NBT + NW - 1) // NW
NPC = N // 16
DRN = 624


def _sc_gather_body(x_hbm, src_hbm, out_hbm, idx_v, rows_v, sem):
    cid = lax.axis_index("c")
    sid = lax.axis_index("s")
    wid = sid * 2 + cid
    start = wid * KF

    def do_batch(b, carry):
        row0 = start + b * KB
        pltpu.sync_copy(src_hbm.at[pl.ds(row0, KB)], idx_v)
        waits = []
        for j in range(KB):
            waits.append(pltpu.async_copy(
                x_hbm.at[idx_v.at[j]], rows_v.at[pl.ds(j * W, W)], sem))
        for wdesc in waits:
            wdesc.wait()
        pltpu.sync_copy(rows_v, out_hbm.at[pl.ds(row0 * W, KB * W)])
        return carry

    lax.fori_loop(0, NB, do_batch, 0)

    @pl.when(wid < TAIL)
    def _():
        row = NW * KF + wid
        pltpu.sync_copy(src_hbm.at[row], idx_v.at[0])
        pltpu.async_copy(x_hbm.at[idx_v.at[0]], rows_v.at[pl.ds(0, W)],
                         sem).wait()
        pltpu.sync_copy(rows_v.at[pl.ds(0, W)], out_hbm.at[pl.ds(row * W, W)])


def _scatter_common(msg_hbm, dst_hbm, zeros_hbm, agg_out, idx_v, rows_v,
                    stage_v, agg_s, sem, extra_row=None, extra_out=None,
                    ones_v=None, cnt_s=None):
    cid = lax.axis_index("c")
    sid = lax.axis_index("s")
    wid = sid * 2 + cid

    pltpu.sync_copy(zeros_hbm, stage_v)
    pltpu.sync_copy(stage_v.at[pl.ds(0, NPC)], agg_s.at[pl.ds(sid * NPC, NPC)])
    if cnt_s is not None:
        pltpu.sync_copy(stage_v.at[pl.ds(0, NPC)],
                        cnt_s.at[pl.ds(sid * NPC, NPC)])
        pltpu.sync_copy(extra_row, ones_v)
    plsc.subcore_barrier()

    def do_batch(b, carry):
        t = b * NW + wid

        @pl.when(t < NBT)
        def _():
            pltpu.sync_copy(dst_hbm.at[pl.ds(t * SKB, SKB)], idx_v)
            pltpu.sync_copy(msg_hbm.at[pl.ds(t * SKB * W, SKB * W)], rows_v)
            waits = []
            for j in range(SKB):
                waits.append(pltpu.async_copy(
                    rows_v.at[pl.ds(j * W, W)], agg_s.at[idx_v.at[j]], sem,
                    add=True))
                if cnt_s is not None:
                    waits.append(pltpu.async_copy(
                        ones_v, cnt_s.at[idx_v.at[j]], sem, add=True))
            for wdesc in waits:
                wdesc.wait()
        return carry

    lax.fori_loop(0, NBATCH, do_batch, 0)

    @pl.when(wid == 0)
    def _():
        pltpu.sync_copy(dst_hbm.at[pl.ds(NBT * SKB, TAILR)],
                        idx_v.at[pl.ds(0, TAILR)])
        pltpu.sync_copy(msg_hbm.at[pl.ds(NBT * SKB * W, TAILR * W)],
                        rows_v.at[pl.ds(0, TAILR * W)])
        waits = []
        for j in range(TAILR):
            waits.append(pltpu.async_copy(
                rows_v.at[pl.ds(j * W, W)], agg_s.at[idx_v.at[j]], sem,
                add=True))
            if cnt_s is not None:
                waits.append(pltpu.async_copy(
                    ones_v, cnt_s.at[idx_v.at[j]], sem, add=True))
        for wdesc in waits:
            wdesc.wait()

    plsc.subcore_barrier()

    def drain(src_s, dst_hbm_out):
        pltpu.sync_copy(src_s.at[pl.ds(sid * DRN, DRN)],
                        stage_v.at[pl.ds(0, DRN)])
        pltpu.sync_copy(stage_v.at[pl.ds(0, DRN)],
                        dst_hbm_out.at[cid, pl.ds(sid * DRN, DRN)])

        @pl.when(sid == 0)
        def _():
            pltpu.sync_copy(src_s.at[pl.ds(16 * DRN, N - 16 * DRN)],
                            stage_v.at[pl.ds(0, N - 16 * DRN)])
            pltpu.sync_copy(stage_v.at[pl.ds(0, N - 16 * DRN)],
                            dst_hbm_out.at[cid, pl.ds(16 * DRN, N - 16 * DRN)])

    drain(agg_s, agg_out)
    if cnt_s is not None:
        drain(cnt_s, extra_out)


def _sc_scatter_cnt_body(msg_hbm, dst_hbm, zeros_hbm, ones_hbm, agg_out,
                         cnt_out, idx_v, rows_v, stage_v, ones_v, agg_s,
                         cnt_s, sem):
    _scatter_common(msg_hbm, dst_hbm, zeros_hbm, agg_out, idx_v, rows_v,
                    stage_v, agg_s, sem, extra_row=ones_hbm,
                    extra_out=cnt_out, ones_v=ones_v, cnt_s=cnt_s)


def _sc_scatter_body(msg_hbm, dst_hbm, zeros_hbm, agg_out, idx_v, rows_v,
                     stage_v, agg_s, sem):
    _scatter_common(msg_hbm, dst_hbm, zeros_hbm, agg_out, idx_v, rows_v,
                    stage_v, agg_s, sem)


import functools


@functools.lru_cache(maxsize=1)
def _sc_kernels():
    mesh = plsc.VectorSubcoreMesh(core_axis_name="c", subcore_axis_name="s")
    params = pltpu.CompilerParams(use_tc_tiling_on_sc=False)
    gather = pl.kernel(
        _sc_gather_body,
        out_type=jax.ShapeDtypeStruct((E, C), jnp.float32),
        mesh=mesh,
        compiler_params=params,
        scratch_types=[
            pltpu.VMEM((KB, W), jnp.int32),
            pltpu.VMEM((KB * W, C), jnp.float32),
            pltpu.SemaphoreType.DMA,
        ],
    )
    scatter_cnt = pl.kernel(
        _sc_scatter_cnt_body,
        compiler_params=params,
        out_type=(
            jax.ShapeDtypeStruct((2, N, C), jnp.float32),
            jax.ShapeDtypeStruct((2, N, C), jnp.float32),
        ),
        mesh=mesh,
        scratch_types=[
            pltpu.VMEM((SKB, W), jnp.int32),
            pltpu.VMEM((SKB * W, C), jnp.float32),
            pltpu.VMEM((NPC + 16, C), jnp.float32),
            pltpu.VMEM((W, C), jnp.float32),
            pltpu.VMEM_SHARED((N, C), jnp.float32),
            pltpu.VMEM_SHARED((N, C), jnp.float32),
            pltpu.SemaphoreType.DMA,
        ],
    )
    scatter = pl.kernel(
        _sc_scatter_body,
        compiler_params=params,
        out_type=jax.ShapeDtypeStruct((2, N, C), jnp.float32),
        mesh=mesh,
        scratch_types=[
            pltpu.VMEM((SKB, W), jnp.int32),
            pltpu.VMEM((SKB * W, C), jnp.float32),
            pltpu.VMEM((NPC + 16, C), jnp.float32),
            pltpu.VMEM_SHARED((N, C), jnp.float32),
            pltpu.SemaphoreType.DMA,
        ],
    )
    return gather, scatter_cnt, scatter


BE = 2560
GE = E // BE


def _tc_msg_body(ea_ref, xs_ref, w1_ref, b1_ref, w2_ref, b2_ref,
                 exp_ref, red_ref, out_ref):
    t = jnp.maximum(
        jnp.dot(ea_ref[...].astype(jnp.bfloat16),
                w1_ref[...].astype(jnp.bfloat16),
                preferred_element_type=jnp.float32) + b1_ref[...], 0.0)
    h = jnp.dot(t.astype(jnp.bfloat16),
                w2_ref[...].astype(jnp.bfloat16),
                preferred_element_type=jnp.float32) + b2_ref[...]
    xs_rep = jnp.dot(xs_ref[...], exp_ref[...],
                     preferred_element_type=jnp.float32)
    out_ref[...] = jnp.dot(xs_rep * h, red_ref[...],
                           preferred_element_type=jnp.float32)


_tc_msg = pl.pallas_call(
    _tc_msg_body,
    grid=(GE,),
    in_specs=[
        pl.BlockSpec((BE, C), lambda i: (i, 0)),
        pl.BlockSpec((BE, C), lambda i: (i, 0)),
        pl.BlockSpec((C, H), lambda i: (0, 0)),
        pl.BlockSpec((1, H), lambda i: (0, 0)),
        pl.BlockSpec((H, H), lambda i: (0, 0)),
        pl.BlockSpec((1, H), lambda i: (0, 0)),
        pl.BlockSpec((C, H), lambda i: (0, 0)),
        pl.BlockSpec((H, C), lambda i: (0, 0)),
    ],
    out_specs=pl.BlockSpec((BE, C), lambda i: (i, 0)),
    out_shape=jax.ShapeDtypeStruct((E, C), jnp.float32),
)


def _tc_combine1_body(x_ref, agg_ref, cnt_ref, root_ref, bias_ref, out_ref):
    aggv = agg_ref[0] + agg_ref[1]
    cntv = jnp.maximum(cnt_ref[0] + cnt_ref[1], 1.0)
    mean = aggv / cntv
    out_ref[...] = jnp.maximum(
        jnp.dot(x_ref[...], root_ref[...], preferred_element_type=jnp.float32)
        + mean + bias_ref[...], 0.0)


_tc_combine1 = pl.pallas_call(
    _tc_combine1_body,
    out_shape=jax.ShapeDtypeStruct((N, C), jnp.float32),
)


def _tc_combine2_body(h_ref, agg_ref, cnt_ref, root_ref, bias_ref, lw_ref,
                      lb_ref, out_ref):
    aggv = agg_ref[0] + agg_ref[1]
    cntv = jnp.maximum(cnt_ref[0] + cnt_ref[1], 1.0)
    h2 = jnp.maximum(
        jnp.dot(h_ref[...], root_ref[...], preferred_element_type=jnp.float32)
        + aggv / cntv + bias_ref[...], 0.0)
    out_ref[...] = jnp.dot(
        h2, lw_ref[...], preferred_element_type=jnp.float32) + lb_ref[...]


_tc_combine2 = pl.pallas_call(
    _tc_combine2_body,
    out_shape=jax.ShapeDtypeStruct((N, NCLS), jnp.float32),
)


def kernel(x, edge_index, edge_attr,
           en1_w1, en1_b1, en1_w2, en1_b2, root1, bias1,
           en2_w1, en2_b1, en2_w2, en2_b2, root2, bias2,
           lin_w, lin_b):
    src2d = edge_index[0].reshape(R, W)
    dst2d = edge_index[1].reshape(R, W)
    zeros = jnp.zeros((NPC + 16, C), jnp.float32)
    ones = jnp.ones((W, C), jnp.float32)
    jidx = jnp.arange(H) // C
    exp_m = (jidx[None, :] == jnp.arange(C)[:, None]).astype(jnp.float32)
    red_m = jnp.tile(jnp.eye(C, dtype=jnp.float32), (C, 1))
    _sc_gather, _sc_scatter_cnt, _sc_scatter = _sc_kernels()

    xs = _sc_gather(x, src2d)
    msg1 = _tc_msg(edge_attr, xs, en1_w1,
                   en1_b1.reshape(1, H), en1_w2, en1_b2.reshape(1, H),
                   exp_m, red_m)
    agg1, cnt = _sc_scatter_cnt(msg1, dst2d, zeros, ones)
    h1 = _tc_combine1(x, agg1, cnt, root1, bias1.reshape(1, C))

    hs = _sc_gather(h1, src2d)
    msg2 = _tc_msg(edge_attr, hs, en2_w1,
                   en2_b1.reshape(1, H), en2_w2, en2_b2.reshape(1, H),
                   exp_m, red_m)
    agg2 = _sc_scatter(msg2, dst2d, zeros)
    return _tc_combine2(h1, agg2, cnt, root2, bias2.reshape(1, C),
                        lin_w, lin_b.reshape(1, NCLS))


# speedup vs baseline: 5.2022x; 1.2714x over previous
import jax
import jax.numpy as jnp
from jax import lax
from jax.experimental import pallas as pl
from jax.experimental.pallas import tpu as pltpu
from jax.experimental.pallas import tpu_sc as plsc

N = 10000
E = 320000
C = 16
H = 256
NCLS = 64
W = 128
R = E // W
NW = 32
KF = R // NW
TAIL = R - KF * NW
KB = 13
NB = KF // KB
SKB = 8
NBT = R // 8
TAILR = R - NBT * 8
NBATCH = (NBT + NW - 1) // NW
NPC = N // 16
DRN = 624


def _sc_gather_body(x_hbm, src_hbm, out_hbm, idx_v, rows_v, sem):
    cid = lax.axis_index("c")
    sid = lax.axis_index("s")
    wid = sid * 2 + cid
    start = wid * KF

    def do_batch(b, carry):
        row0 = start + b * KB
        pltpu.sync_copy(src_hbm.at[pl.ds(row0, KB)], idx_v)
        waits = []
        for j in range(KB):
            waits.append(pltpu.async_copy(
                x_hbm.at[idx_v.at[j]], rows_v.at[pl.ds(j * W, W)], sem))
        for wdesc in waits:
            wdesc.wait()
        pltpu.sync_copy(rows_v, out_hbm.at[pl.ds(row0 * W, KB * W)])
        return carry

    lax.fori_loop(0, NB, do_batch, 0)

    @pl.when(wid < TAIL)
    def _():
        row = NW * KF + wid
        pltpu.sync_copy(src_hbm.at[row], idx_v.at[0])
        pltpu.async_copy(x_hbm.at[idx_v.at[0]], rows_v.at[pl.ds(0, W)],
                         sem).wait()
        pltpu.sync_copy(rows_v.at[pl.ds(0, W)], out_hbm.at[pl.ds(row * W, W)])


def _scatter_common(msg_hbm, dst_hbm, zeros_hbm, agg_out, idx_v, rows_v,
                    stage_v, agg_s, sem, extra_row=None, extra_out=None,
                    ones_v=None, cnt_s=None):
    cid = lax.axis_index("c")
    sid = lax.axis_index("s")
    wid = sid * 2 + cid

    pltpu.sync_copy(zeros_hbm, stage_v)
    pltpu.sync_copy(stage_v.at[pl.ds(0, NPC)], agg_s.at[pl.ds(sid * NPC, NPC)])
    if cnt_s is not None:
        pltpu.sync_copy(stage_v.at[pl.ds(0, NPC)],
                        cnt_s.at[pl.ds(sid * NPC, NPC)])
        pltpu.sync_copy(extra_row, ones_v)
    plsc.subcore_barrier()

    def do_batch(b, carry):
        t = b * NW + wid

        @pl.when(t < NBT)
        def _():
            pltpu.sync_copy(dst_hbm.at[pl.ds(t * SKB, SKB)], idx_v)
            pltpu.sync_copy(msg_hbm.at[pl.ds(t * SKB * W, SKB * W)], rows_v)
            waits = []
            for j in range(SKB):
                waits.append(pltpu.async_copy(
                    rows_v.at[pl.ds(j * W, W)], agg_s.at[idx_v.at[j]], sem,
                    add=True))
                if cnt_s is not None:
                    waits.append(pltpu.async_copy(
                        ones_v, cnt_s.at[idx_v.at[j]], sem, add=True))
            for wdesc in waits:
                wdesc.wait()
        return carry

    lax.fori_loop(0, NBATCH, do_batch, 0)

    @pl.when(wid == 0)
    def _():
        pltpu.sync_copy(dst_hbm.at[pl.ds(NBT * SKB, TAILR)],
                        idx_v.at[pl.ds(0, TAILR)])
        pltpu.sync_copy(msg_hbm.at[pl.ds(NBT * SKB * W, TAILR * W)],
                        rows_v.at[pl.ds(0, TAILR * W)])
        waits = []
        for j in range(TAILR):
            waits.append(pltpu.async_copy(
                rows_v.at[pl.ds(j * W, W)], agg_s.at[idx_v.at[j]], sem,
                add=True))
            if cnt_s is not None:
                waits.append(pltpu.async_copy(
                    ones_v, cnt_s.at[idx_v.at[j]], sem, add=True))
        for wdesc in waits:
            wdesc.wait()

    plsc.subcore_barrier()

    def drain(src_s, dst_hbm_out):
        pltpu.sync_copy(src_s.at[pl.ds(sid * DRN, DRN)],
                        stage_v.at[pl.ds(0, DRN)])
        pltpu.sync_copy(stage_v.at[pl.ds(0, DRN)],
                        dst_hbm_out.at[cid, pl.ds(sid * DRN, DRN)])

        @pl.when(sid == 0)
        def _():
            pltpu.sync_copy(src_s.at[pl.ds(16 * DRN, N - 16 * DRN)],
                            stage_v.at[pl.ds(0, N - 16 * DRN)])
            pltpu.sync_copy(stage_v.at[pl.ds(0, N - 16 * DRN)],
                            dst_hbm_out.at[cid, pl.ds(16 * DRN, N - 16 * DRN)])

    drain(agg_s, agg_out)
    if cnt_s is not None:
        drain(cnt_s, extra_out)


def _sc_scatter_cnt_body(msg_hbm, dst_hbm, zeros_hbm, ones_hbm, agg_out,
                         cnt_out, idx_v, rows_v, stage_v, ones_v, agg_s,
                         cnt_s, sem):
    _scatter_common(msg_hbm, dst_hbm, zeros_hbm, agg_out, idx_v, rows_v,
                    stage_v, agg_s, sem, extra_row=ones_hbm,
                    extra_out=cnt_out, ones_v=ones_v, cnt_s=cnt_s)


def _sc_scatter_body(msg_hbm, dst_hbm, zeros_hbm, agg_out, idx_v, rows_v,
                     stage_v, agg_s, sem):
    _scatter_common(msg_hbm, dst_hbm, zeros_hbm, agg_out, idx_v, rows_v,
                    stage_v, agg_s, sem)


import functools


@functools.lru_cache(maxsize=1)
def _sc_kernels():
    mesh = plsc.VectorSubcoreMesh(core_axis_name="c", subcore_axis_name="s")
    params = pltpu.CompilerParams(use_tc_tiling_on_sc=False)
    gather = pl.kernel(
        _sc_gather_body,
        out_type=jax.ShapeDtypeStruct((E, C), jnp.float32),
        mesh=mesh,
        compiler_params=params,
        scratch_types=[
            pltpu.VMEM((KB, W), jnp.int32),
            pltpu.VMEM((KB * W, C), jnp.float32),
            pltpu.SemaphoreType.DMA,
        ],
    )
    scatter_cnt = pl.kernel(
        _sc_scatter_cnt_body,
        compiler_params=params,
        out_type=(
            jax.ShapeDtypeStruct((2, N, C), jnp.float32),
            jax.ShapeDtypeStruct((2, N, C), jnp.float32),
        ),
        mesh=mesh,
        scratch_types=[
            pltpu.VMEM((SKB, W), jnp.int32),
            pltpu.VMEM((SKB * W, C), jnp.float32),
            pltpu.VMEM((NPC + 16, C), jnp.float32),
            pltpu.VMEM((W, C), jnp.float32),
            pltpu.VMEM_SHARED((N, C), jnp.float32),
            pltpu.VMEM_SHARED((N, C), jnp.float32),
            pltpu.SemaphoreType.DMA,
        ],
    )
    scatter = pl.kernel(
        _sc_scatter_body,
        compiler_params=params,
        out_type=jax.ShapeDtypeStruct((2, N, C), jnp.float32),
        mesh=mesh,
        scratch_types=[
            pltpu.VMEM((SKB, W), jnp.int32),
            pltpu.VMEM((SKB * W, C), jnp.float32),
            pltpu.VMEM((NPC + 16, C), jnp.float32),
            pltpu.VMEM_SHARED((N, C), jnp.float32),
            pltpu.SemaphoreType.DMA,
        ],
    )
    return gather, scatter_cnt, scatter


BE = 2560
GE = E // BE


def _tc_msg_body(ea0, ea1, ea2, ea3, ea4, ea5, ea6, ea7, xsb_ref,
                 w1_ref, b1_ref, w2_ref, b2_ref, exp_ref, redp_ref, out_ref):
    eas = [ea0, ea1, ea2, ea3, ea4, ea5, ea6, ea7]
    xsb = xsb_ref[...]
    acc = jnp.zeros((BE // 8, 128), jnp.float32)
    for a in range(8):
        t = jnp.maximum(
            jnp.dot(eas[a][...].astype(jnp.bfloat16),
                    w1_ref[...].astype(jnp.bfloat16),
                    preferred_element_type=jnp.float32) + b1_ref[...], 0.0)
        h = jnp.dot(t.astype(jnp.bfloat16),
                    w2_ref[...].astype(jnp.bfloat16),
                    preferred_element_type=jnp.float32) + b2_ref[...]
        xs_rep = jnp.dot(xsb[:, 16 * a:16 * (a + 1)], exp_ref[...],
                         preferred_element_type=jnp.float32)
        acc = acc + jnp.dot(xs_rep * h, redp_ref[a],
                            preferred_element_type=jnp.float32)
    out_ref[...] = acc


GEB = E // 8 // (BE // 8)

_tc_msg = pl.pallas_call(
    _tc_msg_body,
    grid=(GEB,),
    in_specs=[
        *[pl.BlockSpec((BE // 8, C), (lambda a: (lambda i: (a * GEB + i, 0)))(a))
          for a in range(8)],
        pl.BlockSpec((BE // 8, 128), lambda i: (i, 0)),
        pl.BlockSpec((C, H), lambda i: (0, 0)),
        pl.BlockSpec((1, H), lambda i: (0, 0)),
        pl.BlockSpec((H, H), lambda i: (0, 0)),
        pl.BlockSpec((1, H), lambda i: (0, 0)),
        pl.BlockSpec((C, H), lambda i: (0, 0)),
        pl.BlockSpec((8, H, 128), lambda i: (0, 0, 0)),
    ],
    out_specs=pl.BlockSpec((BE // 8, 128), lambda i: (i, 0)),
    out_shape=jax.ShapeDtypeStruct((E // 8, 128), jnp.float32),
)


def _tc_combine1_body(x_ref, agg_ref, cnt_ref, root_ref, bias_ref, out_ref):
    aggv = agg_ref[0] + agg_ref[1]
    cntv = jnp.maximum(cnt_ref[0] + cnt_ref[1], 1.0)
    mean = aggv / cntv
    out_ref[...] = jnp.maximum(
        jnp.dot(x_ref[...], root_ref[...], preferred_element_type=jnp.float32)
        + mean + bias_ref[...], 0.0)


_tc_combine1 = pl.pallas_call(
    _tc_combine1_body,
    out_shape=jax.ShapeDtypeStruct((N, C), jnp.float32),
)


def _tc_combine2_body(h_ref, agg_ref, cnt_ref, root_ref, bias_ref, lw_ref,
                      lb_ref, out_ref):
    aggv = agg_ref[0] + agg_ref[1]
    cntv = jnp.maximum(cnt_ref[0] + cnt_ref[1], 1.0)
    h2 = jnp.maximum(
        jnp.dot(h_ref[...], root_ref[...], preferred_element_type=jnp.float32)
        + aggv / cntv + bias_ref[...], 0.0)
    out_ref[...] = jnp.dot(
        h2, lw_ref[...], preferred_element_type=jnp.float32) + lb_ref[...]


_tc_combine2 = pl.pallas_call(
    _tc_combine2_body,
    out_shape=jax.ShapeDtypeStruct((N, NCLS), jnp.float32),
)


def kernel(x, edge_index, edge_attr,
           en1_w1, en1_b1, en1_w2, en1_b2, root1, bias1,
           en2_w1, en2_b1, en2_w2, en2_b2, root2, bias2,
           lin_w, lin_b):
    src_sc = edge_index[0].reshape(8, E // 8).T.reshape(R, W)
    dst_sc = edge_index[1].reshape(8, E // 8).T.reshape(R, W)
    zeros = jnp.zeros((NPC + 16, C), jnp.float32)
    ones = jnp.ones((W, C), jnp.float32)
    jidx = jnp.arange(H) // C
    exp_m = (jidx[None, :] == jnp.arange(C)[:, None]).astype(jnp.float32)
    red_m = jnp.tile(jnp.eye(C, dtype=jnp.float32), (C, 1))
    redp = jnp.zeros((8, H, 128), jnp.float32)
    for a in range(8):
        redp = redp.at[a, :, 16 * a:16 * (a + 1)].set(red_m)
    _sc_gather, _sc_scatter_cnt, _sc_scatter = _sc_kernels()

    xs = _sc_gather(x, src_sc)
    msg1 = _tc_msg(edge_attr, edge_attr, edge_attr, edge_attr, edge_attr,
                   edge_attr, edge_attr, edge_attr,
                   xs.reshape(E // 8, 128), en1_w1, en1_b1.reshape(1, H),
                   en1_w2, en1_b2.reshape(1, H), exp_m, redp)
    agg1, cnt = _sc_scatter_cnt(msg1.reshape(E, C), dst_sc, zeros, ones)
    h1 = _tc_combine1(x, agg1, cnt, root1, bias1.reshape(1, C))

    hs = _sc_gather(h1, src_sc)
    msg2 = _tc_msg(edge_attr, edge_attr, edge_attr, edge_attr, edge_attr,
                   edge_attr, edge_attr, edge_attr,
                   hs.reshape(E // 8, 128), en2_w1, en2_b1.reshape(1, H),
                   en2_w2, en2_b2.reshape(1, H), exp_m, redp)
    agg2 = _sc_scatter(msg2.reshape(E, C), dst_sc, zeros)
    return _tc_combine2(h1, agg2, cnt, root2, bias2.reshape(1, C),
                        lin_w, lin_b.reshape(1, NCLS))


# speedup vs baseline: 6.3335x; 1.2175x over previous
import jax
import jax.numpy as jnp
from jax import lax
from jax.experimental import pallas as pl
from jax.experimental.pallas import tpu as pltpu
from jax.experimental.pallas import tpu_sc as plsc

N = 10000
E = 320000
C = 16
H = 256
NCLS = 64
W = 128
R = E // W
NW = 32
KF = R // NW
TAIL = R - KF * NW
KB = 13
NB = KF // KB
SKB = 8
NBT = R // 8
TAILR = R - NBT * 8
NBATCH = (NBT + NW - 1) // NW
NPC = N // 16
DRN = 624


def _sc_gather_body(x_hbm, src_hbm, out_hbm, idx_v, rows_v, sem):
    cid = lax.axis_index("c")
    sid = lax.axis_index("s")
    wid = sid * 2 + cid
    start = wid * KF

    def do_batch(b, carry):
        row0 = start + b * KB
        pltpu.sync_copy(src_hbm.at[pl.ds(row0, KB)], idx_v)
        waits = []
        for j in range(KB):
            waits.append(pltpu.async_copy(
                x_hbm.at[idx_v.at[j]], rows_v.at[pl.ds(j * W, W)], sem))
        for wdesc in waits:
            wdesc.wait()
        pltpu.sync_copy(rows_v, out_hbm.at[pl.ds(row0 * W, KB * W)])
        return carry

    lax.fori_loop(0, NB, do_batch, 0)

    @pl.when(wid < TAIL)
    def _():
        row = NW * KF + wid
        pltpu.sync_copy(src_hbm.at[row], idx_v.at[0])
        pltpu.async_copy(x_hbm.at[idx_v.at[0]], rows_v.at[pl.ds(0, W)],
                         sem).wait()
        pltpu.sync_copy(rows_v.at[pl.ds(0, W)], out_hbm.at[pl.ds(row * W, W)])


def _scatter_common(msg_hbm, dst_hbm, zeros_hbm, agg_out, idx_v, rows_v,
                    stage_v, agg_s, sem, extra_row=None, extra_out=None,
                    ones_v=None, cnt_s=None):
    cid = lax.axis_index("c")
    sid = lax.axis_index("s")
    wid = sid * 2 + cid

    pltpu.sync_copy(zeros_hbm, stage_v)
    pltpu.sync_copy(stage_v.at[pl.ds(0, NPC)], agg_s.at[pl.ds(sid * NPC, NPC)])
    if cnt_s is not None:
        pltpu.sync_copy(stage_v.at[pl.ds(0, NPC)],
                        cnt_s.at[pl.ds(sid * NPC, NPC)])
        pltpu.sync_copy(extra_row, ones_v)
    plsc.subcore_barrier()

    def do_batch(b, carry):
        t = b * NW + wid

        @pl.when(t < NBT)
        def _():
            pltpu.sync_copy(dst_hbm.at[pl.ds(t * SKB, SKB)], idx_v)
            pltpu.sync_copy(msg_hbm.at[pl.ds(t * SKB * W, SKB * W)], rows_v)
            waits = []
            for j in range(SKB):
                waits.append(pltpu.async_copy(
                    rows_v.at[pl.ds(j * W, W)], agg_s.at[idx_v.at[j]], sem,
                    add=True))
                if cnt_s is not None:
                    waits.append(pltpu.async_copy(
                        ones_v, cnt_s.at[idx_v.at[j]], sem, add=True))
            for wdesc in waits:
                wdesc.wait()
        return carry

    lax.fori_loop(0, NBATCH, do_batch, 0)

    @pl.when(wid == 0)
    def _():
        pltpu.sync_copy(dst_hbm.at[pl.ds(NBT * SKB, TAILR)],
                        idx_v.at[pl.ds(0, TAILR)])
        pltpu.sync_copy(msg_hbm.at[pl.ds(NBT * SKB * W, TAILR * W)],
                        rows_v.at[pl.ds(0, TAILR * W)])
        waits = []
        for j in range(TAILR):
            waits.append(pltpu.async_copy(
                rows_v.at[pl.ds(j * W, W)], agg_s.at[idx_v.at[j]], sem,
                add=True))
            if cnt_s is not None:
                waits.append(pltpu.async_copy(
                    ones_v, cnt_s.at[idx_v.at[j]], sem, add=True))
        for wdesc in waits:
            wdesc.wait()

    plsc.subcore_barrier()

    def drain(src_s, dst_hbm_out):
        pltpu.sync_copy(src_s.at[pl.ds(sid * DRN, DRN)],
                        stage_v.at[pl.ds(0, DRN)])
        pltpu.sync_copy(stage_v.at[pl.ds(0, DRN)],
                        dst_hbm_out.at[cid, pl.ds(sid * DRN, DRN)])

        @pl.when(sid == 0)
        def _():
            pltpu.sync_copy(src_s.at[pl.ds(16 * DRN, N - 16 * DRN)],
                            stage_v.at[pl.ds(0, N - 16 * DRN)])
            pltpu.sync_copy(stage_v.at[pl.ds(0, N - 16 * DRN)],
                            dst_hbm_out.at[cid, pl.ds(16 * DRN, N - 16 * DRN)])

    drain(agg_s, agg_out)
    if cnt_s is not None:
        drain(cnt_s, extra_out)


def _sc_scatter_cnt_body(msg_hbm, dst_hbm, zeros_hbm, ones_hbm, agg_out,
                         cnt_out, idx_v, rows_v, stage_v, ones_v, agg_s,
                         cnt_s, sem):
    _scatter_common(msg_hbm, dst_hbm, zeros_hbm, agg_out, idx_v, rows_v,
                    stage_v, agg_s, sem, extra_row=ones_hbm,
                    extra_out=cnt_out, ones_v=ones_v, cnt_s=cnt_s)


def _sc_scatter_body(msg_hbm, dst_hbm, zeros_hbm, agg_out, idx_v, rows_v,
                     stage_v, agg_s, sem):
    _scatter_common(msg_hbm, dst_hbm, zeros_hbm, agg_out, idx_v, rows_v,
                    stage_v, agg_s, sem)


import functools


@functools.lru_cache(maxsize=1)
def _sc_kernels():
    mesh = plsc.VectorSubcoreMesh(core_axis_name="c", subcore_axis_name="s")
    params = pltpu.CompilerParams(use_tc_tiling_on_sc=False)
    gather = pl.kernel(
        _sc_gather_body,
        out_type=jax.ShapeDtypeStruct((E, C), jnp.float32),
        mesh=mesh,
        compiler_params=params,
        scratch_types=[
            pltpu.VMEM((KB, W), jnp.int32),
            pltpu.VMEM((KB * W, C), jnp.float32),
            pltpu.SemaphoreType.DMA,
        ],
    )
    scatter_cnt = pl.kernel(
        _sc_scatter_cnt_body,
        compiler_params=params,
        out_type=(
            jax.ShapeDtypeStruct((2, N, C), jnp.float32),
            jax.ShapeDtypeStruct((2, N, C), jnp.float32),
        ),
        mesh=mesh,
        scratch_types=[
            pltpu.VMEM((SKB, W), jnp.int32),
            pltpu.VMEM((SKB * W, C), jnp.float32),
            pltpu.VMEM((NPC + 16, C), jnp.float32),
            pltpu.VMEM((W, C), jnp.float32),
            pltpu.VMEM_SHARED((N, C), jnp.float32),
            pltpu.VMEM_SHARED((N, C), jnp.float32),
            pltpu.SemaphoreType.DMA,
        ],
    )
    scatter = pl.kernel(
        _sc_scatter_body,
        compiler_params=params,
        out_type=jax.ShapeDtypeStruct((2, N, C), jnp.float32),
        mesh=mesh,
        scratch_types=[
            pltpu.VMEM((SKB, W), jnp.int32),
            pltpu.VMEM((SKB * W, C), jnp.float32),
            pltpu.VMEM((NPC + 16, C), jnp.float32),
            pltpu.VMEM_SHARED((N, C), jnp.float32),
            pltpu.SemaphoreType.DMA,
        ],
    )
    return gather, scatter_cnt, scatter


BE = 6400
GE = E // BE


def _tc_msg_body(ea0, ea1, ea2, ea3, ea4, ea5, ea6, ea7, xsb_ref,
                 w1_ref, b1_ref, w2_ref, b2_ref, exp_ref, redp_ref, out_ref):
    eas = [ea0, ea1, ea2, ea3, ea4, ea5, ea6, ea7]
    xsb = xsb_ref[...]
    acc = jnp.zeros((BE // 8, 128), jnp.float32)
    for a in range(8):
        t = jnp.maximum(
            jnp.dot(eas[a][...].astype(jnp.bfloat16),
                    w1_ref[...].astype(jnp.bfloat16),
                    preferred_element_type=jnp.float32) + b1_ref[...], 0.0)
        h = jnp.dot(t.astype(jnp.bfloat16),
                    w2_ref[...].astype(jnp.bfloat16),
                    preferred_element_type=jnp.float32) + b2_ref[...]
        xs_rep = jnp.dot(xsb[:, 16 * a:16 * (a + 1)], exp_ref[...],
                         preferred_element_type=jnp.float32)
        acc = acc + jnp.dot(xs_rep * h, redp_ref[a],
                            preferred_element_type=jnp.float32)
    out_ref[...] = acc


GEB = E // 8 // (BE // 8)

_tc_msg = pl.pallas_call(
    _tc_msg_body,
    grid=(GEB,),
    in_specs=[
        *[pl.BlockSpec((BE // 8, C), (lambda a: (lambda i: (a * GEB + i, 0)))(a))
          for a in range(8)],
        pl.BlockSpec((BE // 8, 128), lambda i: (i, 0)),
        pl.BlockSpec((C, H), lambda i: (0, 0)),
        pl.BlockSpec((1, H), lambda i: (0, 0)),
        pl.BlockSpec((H, H), lambda i: (0, 0)),
        pl.BlockSpec((1, H), lambda i: (0, 0)),
        pl.BlockSpec((C, H), lambda i: (0, 0)),
        pl.BlockSpec((8, H, 128), lambda i: (0, 0, 0)),
    ],
    out_specs=pl.BlockSpec((BE // 8, 128), lambda i: (i, 0)),
    out_shape=jax.ShapeDtypeStruct((E // 8, 128), jnp.float32),
)


def _tc_combine1_body(x_ref, agg_ref, cnt_ref, root_ref, bias_ref, out_ref):
    aggv = agg_ref[0] + agg_ref[1]
    cntv = jnp.maximum(cnt_ref[0] + cnt_ref[1], 1.0)
    mean = aggv / cntv
    out_ref[...] = jnp.maximum(
        jnp.dot(x_ref[...], root_ref[...], preferred_element_type=jnp.float32)
        + mean + bias_ref[...], 0.0)


_tc_combine1 = pl.pallas_call(
    _tc_combine1_body,
    out_shape=jax.ShapeDtypeStruct((N, C), jnp.float32),
)


def _tc_combine2_body(h_ref, agg_ref, cnt_ref, root_ref, bias_ref, lw_ref,
                      lb_ref, out_ref):
    aggv = agg_ref[0] + agg_ref[1]
    cntv = jnp.maximum(cnt_ref[0] + cnt_ref[1], 1.0)
    h2 = jnp.maximum(
        jnp.dot(h_ref[...], root_ref[...], preferred_element_type=jnp.float32)
        + aggv / cntv + bias_ref[...], 0.0)
    out_ref[...] = jnp.dot(
        h2, lw_ref[...], preferred_element_type=jnp.float32) + lb_ref[...]


_tc_combine2 = pl.pallas_call(
    _tc_combine2_body,
    out_shape=jax.ShapeDtypeStruct((N, NCLS), jnp.float32),
)


def kernel(x, edge_index, edge_attr,
           en1_w1, en1_b1, en1_w2, en1_b2, root1, bias1,
           en2_w1, en2_b1, en2_w2, en2_b2, root2, bias2,
           lin_w, lin_b):
    src_sc = edge_index[0].reshape(8, E // 8).T.reshape(R, W)
    dst_sc = edge_index[1].reshape(8, E // 8).T.reshape(R, W)
    zeros = jnp.zeros((NPC + 16, C), jnp.float32)
    ones = jnp.ones((W, C), jnp.float32)
    jidx = jnp.arange(H) // C
    exp_m = (jidx[None, :] == jnp.arange(C)[:, None]).astype(jnp.float32)
    la = jnp.arange(128)
    redp = ((la[None, None, :] // C == jnp.arange(8)[:, None, None])
            & (jnp.arange(H)[None, :, None] % C == la[None, None, :] % C)
            ).astype(jnp.float32)
    _sc_gather, _sc_scatter_cnt, _sc_scatter = _sc_kernels()

    xs = _sc_gather(x, src_sc)
    msg1 = _tc_msg(edge_attr, edge_attr, edge_attr, edge_attr, edge_attr,
                   edge_attr, edge_attr, edge_attr,
                   xs.reshape(E // 8, 128), en1_w1, en1_b1.reshape(1, H),
                   en1_w2, en1_b2.reshape(1, H), exp_m, redp)
    agg1, cnt = _sc_scatter_cnt(msg1.reshape(E, C), dst_sc, zeros, ones)
    h1 = _tc_combine1(x, agg1, cnt, root1, bias1.reshape(1, C))

    hs = _sc_gather(h1, src_sc)
    msg2 = _tc_msg(edge_attr, edge_attr, edge_attr, edge_attr, edge_attr,
                   edge_attr, edge_attr, edge_attr,
                   hs.reshape(E // 8, 128), en2_w1, en2_b1.reshape(1, H),
                   en2_w2, en2_b2.reshape(1, H), exp_m, redp)
    agg2 = _sc_scatter(msg2.reshape(E, C), dst_sc, zeros)
    return _tc_combine2(h1, agg2, cnt, root2, bias2.reshape(1, C),
                        lin_w, lin_b.reshape(1, NCLS))


# speedup vs baseline: 6.5392x; 1.0325x over previous
import jax
import jax.numpy as jnp
from jax import lax
from jax.experimental import pallas as pl
from jax.experimental.pallas import tpu as pltpu
from jax.experimental.pallas import tpu_sc as plsc

N = 10000
E = 320000
C = 16
H = 256
NCLS = 64
W = 128
R = E // W
NW = 32
KF = R // NW
TAIL = R - KF * NW
KB = 26
NB = KF // KB
SKB = 13
SNB = KF // SKB
NPC = N // 16


def _sc_gather_body(x_hbm, src_hbm, out_hbm, idx_v, rows_v, semi, semg,
                    semw):
    cid = lax.axis_index("c")
    sid = lax.axis_index("s")
    wid = sid * 2 + cid
    start = wid * KF

    @pl.when(wid < TAIL)
    def _():
        row = NW * KF + wid
        pltpu.sync_copy(src_hbm.at[row], idx_v.at[0, 0])
        pltpu.async_copy(x_hbm.at[idx_v.at[0, 0]],
                         rows_v.at[0, pl.ds(0, W)], semg).wait()
        pltpu.sync_copy(rows_v.at[0, pl.ds(0, W)],
                        out_hbm.at[pl.ds(row * W, W)])

    didx = {}
    dw = {}
    didx[0] = pltpu.async_copy(src_hbm.at[pl.ds(start, KB)], idx_v.at[0],
                               semi)
    for b in range(NB):
        p = b % 2
        didx[b].wait()
        if b >= 2:
            dw[b - 2].wait()
        gd = []
        for j in range(KB):
            gd.append(pltpu.async_copy(
                x_hbm.at[idx_v.at[p, j]], rows_v.at[p, pl.ds(j * W, W)],
                semg))
        if b + 1 < NB:
            didx[b + 1] = pltpu.async_copy(
                src_hbm.at[pl.ds(start + (b + 1) * KB, KB)],
                idx_v.at[(b + 1) % 2], semi)
        for g in gd:
            g.wait()
        dw[b] = pltpu.async_copy(
            rows_v.at[p], out_hbm.at[pl.ds((start + b * KB) * W, KB * W)],
            semw)
    for b in range(max(0, NB - 2), NB):
        dw[b].wait()


def _scatter_common(msg_hbm, dst_hbm, zeros_hbm, agg_out, idx_v, rows_v,
                    stage_v, agg_s, semi, semr, sema, extra_row=None,
                    extra_out=None, ones_v=None, cnt_s=None):
    cid = lax.axis_index("c")
    sid = lax.axis_index("s")
    wid = sid * 2 + cid
    start = wid * KF

    pltpu.sync_copy(zeros_hbm, stage_v)
    pltpu.sync_copy(stage_v, agg_s.at[pl.ds(sid * NPC, NPC)])
    if cnt_s is not None:
        pltpu.sync_copy(stage_v, cnt_s.at[pl.ds(sid * NPC, NPC)])
        pltpu.sync_copy(extra_row, ones_v)
    plsc.subcore_barrier()

    didx = {}
    dmsg = {}
    adds = {0: [], 1: []}
    didx[0] = pltpu.async_copy(dst_hbm.at[pl.ds(start, SKB)], idx_v.at[0],
                               semi)
    dmsg[0] = pltpu.async_copy(msg_hbm.at[pl.ds(start * W, SKB * W)],
                               rows_v.at[0], semr)
    for b in range(SNB):
        p = b % 2
        didx[b].wait()
        dmsg[b].wait()
        if b >= 2:
            for a in adds[p]:
                a.wait()
        adds[p] = []
        for j in range(SKB):
            adds[p].append(pltpu.async_copy(
                rows_v.at[p, pl.ds(j * W, W)], agg_s.at[idx_v.at[p, j]],
                sema, add=True))
            if cnt_s is not None:
                adds[p].append(pltpu.async_copy(
                    ones_v, cnt_s.at[idx_v.at[p, j]], sema, add=True))
        if b + 1 < SNB:
            row0 = start + (b + 1) * SKB
            didx[b + 1] = pltpu.async_copy(
                dst_hbm.at[pl.ds(row0, SKB)], idx_v.at[(b + 1) % 2], semi)
            dmsg[b + 1] = pltpu.async_copy(
                msg_hbm.at[pl.ds(row0 * W, SKB * W)],
                rows_v.at[(b + 1) % 2], semr)
    for p in (0, 1):
        for a in adds[p]:
            a.wait()

    @pl.when(wid < TAIL)
    def _():
        row = NW * KF + wid
        pltpu.sync_copy(dst_hbm.at[row], idx_v.at[0, 0])
        pltpu.sync_copy(msg_hbm.at[pl.ds(row * W, W)],
                        rows_v.at[0, pl.ds(0, W)])
        pltpu.sync_copy(rows_v.at[0, pl.ds(0, W)], agg_s.at[idx_v.at[0, 0]],
                        add=True)
        if cnt_s is not None:
            pltpu.sync_copy(ones_v, cnt_s.at[idx_v.at[0, 0]], add=True)

    plsc.subcore_barrier()

    def drain(src_s, dst_hbm_out):
        pltpu.sync_copy(src_s.at[pl.ds(sid * NPC, NPC)], stage_v)
        pltpu.sync_copy(stage_v, dst_hbm_out.at[cid, pl.ds(sid * NPC, NPC)])

    drain(agg_s, agg_out)
    if cnt_s is not None:
        drain(cnt_s, extra_out)


def _sc_scatter_cnt_body(msg_hbm, dst_hbm, zeros_hbm, ones_hbm, agg_out,
                         cnt_out, idx_v, rows_v, stage_v, ones_v, agg_s,
                         cnt_s, semi, semr, sema):
    _scatter_common(msg_hbm, dst_hbm, zeros_hbm, agg_out, idx_v, rows_v,
                    stage_v, agg_s, semi, semr, sema, extra_row=ones_hbm,
                    extra_out=cnt_out, ones_v=ones_v, cnt_s=cnt_s)


def _sc_scatter_body(msg_hbm, dst_hbm, zeros_hbm, agg_out, idx_v, rows_v,
                     stage_v, agg_s, semi, semr, sema):
    _scatter_common(msg_hbm, dst_hbm, zeros_hbm, agg_out, idx_v, rows_v,
                    stage_v, agg_s, semi, semr, sema)


import functools


@functools.lru_cache(maxsize=1)
def _sc_kernels():
    mesh = plsc.VectorSubcoreMesh(core_axis_name="c", subcore_axis_name="s")
    params = pltpu.CompilerParams(use_tc_tiling_on_sc=False)
    gather = pl.kernel(
        _sc_gather_body,
        out_type=jax.ShapeDtypeStruct((E, C), jnp.float32),
        mesh=mesh,
        compiler_params=params,
        scratch_types=[
            pltpu.VMEM((2, KB, W), jnp.int32),
            pltpu.VMEM((2, KB * W, C), jnp.float32),
            pltpu.SemaphoreType.DMA,
            pltpu.SemaphoreType.DMA,
            pltpu.SemaphoreType.DMA,
        ],
    )
    scatter_cnt = pl.kernel(
        _sc_scatter_cnt_body,
        compiler_params=params,
        out_type=(
            jax.ShapeDtypeStruct((2, N, C), jnp.float32),
            jax.ShapeDtypeStruct((2, N, C), jnp.float32),
        ),
        mesh=mesh,
        scratch_types=[
            pltpu.VMEM((2, SKB, W), jnp.int32),
            pltpu.VMEM((2, SKB * W, C), jnp.float32),
            pltpu.VMEM((NPC, C), jnp.float32),
            pltpu.VMEM((W, C), jnp.float32),
            pltpu.VMEM_SHARED((N, C), jnp.float32),
            pltpu.VMEM_SHARED((N, C), jnp.float32),
            pltpu.SemaphoreType.DMA,
            pltpu.SemaphoreType.DMA,
            pltpu.SemaphoreType.DMA,
        ],
    )
    scatter = pl.kernel(
        _sc_scatter_body,
        compiler_params=params,
        out_type=jax.ShapeDtypeStruct((2, N, C), jnp.float32),
        mesh=mesh,
        scratch_types=[
            pltpu.VMEM((2, SKB, W), jnp.int32),
            pltpu.VMEM((2, SKB * W, C), jnp.float32),
            pltpu.VMEM((NPC, C), jnp.float32),
            pltpu.VMEM_SHARED((N, C), jnp.float32),
            pltpu.SemaphoreType.DMA,
            pltpu.SemaphoreType.DMA,
            pltpu.SemaphoreType.DMA,
        ],
    )
    return gather, scatter_cnt, scatter


BE = 6400
GE = E // BE


def _tc_msg_body(ea0, ea1, ea2, ea3, ea4, ea5, ea6, ea7, xsb_ref,
                 w1_ref, b1_ref, w2_ref, b2_ref, exp_ref, redp_ref, out_ref):
    eas = [ea0, ea1, ea2, ea3, ea4, ea5, ea6, ea7]
    xsb = xsb_ref[...]
    acc = jnp.zeros((BE // 8, 128), jnp.float32)
    for a in range(8):
        t = jnp.maximum(
            jnp.dot(eas[a][...].astype(jnp.bfloat16),
                    w1_ref[...].astype(jnp.bfloat16),
                    preferred_element_type=jnp.float32) + b1_ref[...], 0.0)
        h = jnp.dot(t.astype(jnp.bfloat16),
                    w2_ref[...].astype(jnp.bfloat16),
                    preferred_element_type=jnp.float32) + b2_ref[...]
        xs_rep = jnp.dot(xsb[:, 16 * a:16 * (a + 1)], exp_ref[...],
                         preferred_element_type=jnp.float32)
        acc = acc + jnp.dot(xs_rep * h, redp_ref[a],
                            preferred_element_type=jnp.float32)
    out_ref[...] = acc


GEB = E // 8 // (BE // 8)

_tc_msg = pl.pallas_call(
    _tc_msg_body,
    grid=(GEB,),
    in_specs=[
        *[pl.BlockSpec((BE // 8, C), (lambda a: (lambda i: (a * GEB + i, 0)))(a))
          for a in range(8)],
        pl.BlockSpec((BE // 8, 128), lambda i: (i, 0)),
        pl.BlockSpec((C, H), lambda i: (0, 0)),
        pl.BlockSpec((1, H), lambda i: (0, 0)),
        pl.BlockSpec((H, H), lambda i: (0, 0)),
        pl.BlockSpec((1, H), lambda i: (0, 0)),
        pl.BlockSpec((C, H), lambda i: (0, 0)),
        pl.BlockSpec((8, H, 128), lambda i: (0, 0, 0)),
    ],
    out_specs=pl.BlockSpec((BE // 8, 128), lambda i: (i, 0)),
    out_shape=jax.ShapeDtypeStruct((E // 8, 128), jnp.float32),
)


def _tc_combine1_body(x_ref, agg_ref, cnt_ref, root_ref, bias_ref, out_ref):
    aggv = agg_ref[0] + agg_ref[1]
    cntv = jnp.maximum(cnt_ref[0] + cnt_ref[1], 1.0)
    mean = aggv / cntv
    out_ref[...] = jnp.maximum(
        jnp.dot(x_ref[...], root_ref[...], preferred_element_type=jnp.float32)
        + mean + bias_ref[...], 0.0)


_tc_combine1 = pl.pallas_call(
    _tc_combine1_body,
    out_shape=jax.ShapeDtypeStruct((N, C), jnp.float32),
)


def _tc_combine2_body(h_ref, agg_ref, cnt_ref, root_ref, bias_ref, lw_ref,
                      lb_ref, out_ref):
    aggv = agg_ref[0] + agg_ref[1]
    cntv = jnp.maximum(cnt_ref[0] + cnt_ref[1], 1.0)
    h2 = jnp.maximum(
        jnp.dot(h_ref[...], root_ref[...], preferred_element_type=jnp.float32)
        + aggv / cntv + bias_ref[...], 0.0)
    out_ref[...] = jnp.dot(
        h2, lw_ref[...], preferred_element_type=jnp.float32) + lb_ref[...]


_tc_combine2 = pl.pallas_call(
    _tc_combine2_body,
    out_shape=jax.ShapeDtypeStruct((N, NCLS), jnp.float32),
)


def kernel(x, edge_index, edge_attr,
           en1_w1, en1_b1, en1_w2, en1_b2, root1, bias1,
           en2_w1, en2_b1, en2_w2, en2_b2, root2, bias2,
           lin_w, lin_b):
    src_sc = edge_index[0].reshape(8, E // 8).T.reshape(R, W)
    dst_sc = edge_index[1].reshape(8, E // 8).T.reshape(R, W)
    zeros = jnp.zeros((NPC, C), jnp.float32)
    ones = jnp.ones((W, C), jnp.float32)
    jidx = jnp.arange(H) // C
    exp_m = (jidx[None, :] == jnp.arange(C)[:, None]).astype(jnp.float32)
    la = jnp.arange(128)
    redp = ((la[None, None, :] // C == jnp.arange(8)[:, None, None])
            & (jnp.arange(H)[None, :, None] % C == la[None, None, :] % C)
            ).astype(jnp.float32)
    _sc_gather, _sc_scatter_cnt, _sc_scatter = _sc_kernels()

    xs = _sc_gather(x, src_sc)
    msg1 = _tc_msg(edge_attr, edge_attr, edge_attr, edge_attr, edge_attr,
                   edge_attr, edge_attr, edge_attr,
                   xs.reshape(E // 8, 128), en1_w1, en1_b1.reshape(1, H),
                   en1_w2, en1_b2.reshape(1, H), exp_m, redp)
    agg1, cnt = _sc_scatter_cnt(msg1.reshape(E, C), dst_sc, zeros, ones)
    h1 = _tc_combine1(x, agg1, cnt, root1, bias1.reshape(1, C))

    hs = _sc_gather(h1, src_sc)
    msg2 = _tc_msg(edge_attr, edge_attr, edge_attr, edge_attr, edge_attr,
                   edge_attr, edge_attr, edge_attr,
                   hs.reshape(E // 8, 128), en2_w1, en2_b1.reshape(1, H),
                   en2_w2, en2_b2.reshape(1, H), exp_m, redp)
    agg2 = _sc_scatter(msg2.reshape(E, C), dst_sc, zeros)
    return _tc_combine2(h1, agg2, cnt, root2, bias2.reshape(1, C),
                        lin_w, lin_b.reshape(1, NCLS))


# speedup vs baseline: 6.9313x; 1.0600x over previous
import jax
import jax.numpy as jnp
from jax import lax
from jax.experimental import pallas as pl
from jax.experimental.pallas import tpu as pltpu
from jax.experimental.pallas import tpu_sc as plsc

N = 10000
E = 320000
C = 16
H = 256
NCLS = 64
W = 128
R = E // W
NW = 32
KF = R // NW
TAIL = R - KF * NW
KB = 26
NB = KF // KB
SKB = 13
SNB = KF // SKB
NPC = N // 16


def _sc_gather_body(x_hbm, src_hbm, out_hbm, idx_v, rows_v, semi, semg,
                    semw):
    cid = lax.axis_index("c")
    sid = lax.axis_index("s")
    wid = sid * 2 + cid
    start = wid * KF

    @pl.when(wid < TAIL)
    def _():
        row = NW * KF + wid
        pltpu.sync_copy(src_hbm.at[row], idx_v.at[0, 0])
        pltpu.async_copy(x_hbm.at[idx_v.at[0, 0]],
                         rows_v.at[0, pl.ds(0, W)], semg).wait()
        pltpu.sync_copy(rows_v.at[0, pl.ds(0, W)],
                        out_hbm.at[pl.ds(row * W, W)])

    didx = {}
    dw = {}
    didx[0] = pltpu.async_copy(src_hbm.at[pl.ds(start, KB)], idx_v.at[0],
                               semi)
    for b in range(NB):
        p = b % 2
        didx[b].wait()
        if b >= 2:
            dw[b - 2].wait()
        gd = []
        for j in range(KB):
            gd.append(pltpu.async_copy(
                x_hbm.at[idx_v.at[p, j]], rows_v.at[p, pl.ds(j * W, W)],
                semg))
        if b + 1 < NB:
            didx[b + 1] = pltpu.async_copy(
                src_hbm.at[pl.ds(start + (b + 1) * KB, KB)],
                idx_v.at[(b + 1) % 2], semi)
        for g in gd:
            g.wait()
        dw[b] = pltpu.async_copy(
            rows_v.at[p], out_hbm.at[pl.ds((start + b * KB) * W, KB * W)],
            semw)
    for b in range(max(0, NB - 2), NB):
        dw[b].wait()


def _scatter_common(msg_hbm, dst_hbm, zeros_hbm, agg_out, idx_v, rows_v,
                    stage_v, agg_s, semi, semr, sema, extra_row=None,
                    extra_out=None, ones_v=None, cnt_s=None):
    cid = lax.axis_index("c")
    sid = lax.axis_index("s")
    wid = sid * 2 + cid
    start = wid * KF

    pltpu.sync_copy(zeros_hbm, stage_v)
    pltpu.sync_copy(stage_v, agg_s.at[pl.ds(sid * NPC, NPC)])
    if cnt_s is not None:
        pltpu.sync_copy(stage_v, cnt_s.at[pl.ds(sid * NPC, NPC)])
        pltpu.sync_copy(extra_row, ones_v)
    plsc.subcore_barrier()

    didx = {}
    dmsg = {}
    adds = {0: [], 1: []}
    didx[0] = pltpu.async_copy(dst_hbm.at[pl.ds(start, SKB)], idx_v.at[0],
                               semi)
    dmsg[0] = pltpu.async_copy(msg_hbm.at[pl.ds(start * W, SKB * W)],
                               rows_v.at[0], semr)
    for b in range(SNB):
        p = b % 2
        didx[b].wait()
        dmsg[b].wait()
        if b >= 2:
            for a in adds[p]:
                a.wait()
        adds[p] = []
        for j in range(SKB):
            adds[p].append(pltpu.async_copy(
                rows_v.at[p, pl.ds(j * W, W)], agg_s.at[idx_v.at[p, j]],
                sema, add=True))
            if cnt_s is not None:
                adds[p].append(pltpu.async_copy(
                    ones_v, cnt_s.at[idx_v.at[p, j]], sema, add=True))
        if b + 1 < SNB:
            row0 = start + (b + 1) * SKB
            didx[b + 1] = pltpu.async_copy(
                dst_hbm.at[pl.ds(row0, SKB)], idx_v.at[(b + 1) % 2], semi)
            dmsg[b + 1] = pltpu.async_copy(
                msg_hbm.at[pl.ds(row0 * W, SKB * W)],
                rows_v.at[(b + 1) % 2], semr)
    for p in (0, 1):
        for a in adds[p]:
            a.wait()

    @pl.when(wid < TAIL)
    def _():
        row = NW * KF + wid
        pltpu.sync_copy(dst_hbm.at[row], idx_v.at[0, 0])
        pltpu.sync_copy(msg_hbm.at[pl.ds(row * W, W)],
                        rows_v.at[0, pl.ds(0, W)])
        pltpu.sync_copy(rows_v.at[0, pl.ds(0, W)], agg_s.at[idx_v.at[0, 0]],
                        add=True)
        if cnt_s is not None:
            pltpu.sync_copy(ones_v, cnt_s.at[idx_v.at[0, 0]], add=True)

    plsc.subcore_barrier()

    def drain(src_s, dst_hbm_out):
        pltpu.sync_copy(src_s.at[pl.ds(sid * NPC, NPC)], stage_v)
        pltpu.sync_copy(stage_v, dst_hbm_out.at[cid, pl.ds(sid * NPC, NPC)])

    drain(agg_s, agg_out)
    if cnt_s is not None:
        drain(cnt_s, extra_out)


def _sc_scatter_cnt_body(msg_hbm, dst_hbm, zeros_hbm, ones_hbm, agg_out,
                         cnt_out, idx_v, rows_v, stage_v, ones_v, agg_s,
                         cnt_s, semi, semr, sema):
    _scatter_common(msg_hbm, dst_hbm, zeros_hbm, agg_out, idx_v, rows_v,
                    stage_v, agg_s, semi, semr, sema, extra_row=ones_hbm,
                    extra_out=cnt_out, ones_v=ones_v, cnt_s=cnt_s)


def _sc_scatter_body(msg_hbm, dst_hbm, zeros_hbm, agg_out, idx_v, rows_v,
                     stage_v, agg_s, semi, semr, sema):
    _scatter_common(msg_hbm, dst_hbm, zeros_hbm, agg_out, idx_v, rows_v,
                    stage_v, agg_s, semi, semr, sema)


import functools


@functools.lru_cache(maxsize=1)
def _sc_kernels():
    mesh = plsc.VectorSubcoreMesh(core_axis_name="c", subcore_axis_name="s")
    params = pltpu.CompilerParams(use_tc_tiling_on_sc=False)
    gather = pl.kernel(
        _sc_gather_body,
        out_type=jax.ShapeDtypeStruct((E, C), jnp.float32),
        mesh=mesh,
        compiler_params=params,
        scratch_types=[
            pltpu.VMEM((2, KB, W), jnp.int32),
            pltpu.VMEM((2, KB * W, C), jnp.float32),
            pltpu.SemaphoreType.DMA,
            pltpu.SemaphoreType.DMA,
            pltpu.SemaphoreType.DMA,
        ],
    )
    scatter_cnt = pl.kernel(
        _sc_scatter_cnt_body,
        compiler_params=params,
        out_type=(
            jax.ShapeDtypeStruct((2, N, C), jnp.float32),
            jax.ShapeDtypeStruct((2, N, C), jnp.float32),
        ),
        mesh=mesh,
        scratch_types=[
            pltpu.VMEM((2, SKB, W), jnp.int32),
            pltpu.VMEM((2, SKB * W, C), jnp.float32),
            pltpu.VMEM((NPC, C), jnp.float32),
            pltpu.VMEM((W, C), jnp.float32),
            pltpu.VMEM_SHARED((N, C), jnp.float32),
            pltpu.VMEM_SHARED((N, C), jnp.float32),
            pltpu.SemaphoreType.DMA,
            pltpu.SemaphoreType.DMA,
            pltpu.SemaphoreType.DMA,
        ],
    )
    scatter = pl.kernel(
        _sc_scatter_body,
        compiler_params=params,
        out_type=jax.ShapeDtypeStruct((2, N, C), jnp.float32),
        mesh=mesh,
        scratch_types=[
            pltpu.VMEM((2, SKB, W), jnp.int32),
            pltpu.VMEM((2, SKB * W, C), jnp.float32),
            pltpu.VMEM((NPC, C), jnp.float32),
            pltpu.VMEM_SHARED((N, C), jnp.float32),
            pltpu.SemaphoreType.DMA,
            pltpu.SemaphoreType.DMA,
            pltpu.SemaphoreType.DMA,
        ],
    )
    return gather, scatter_cnt, scatter


BE = 6400
GE = E // BE


def _tc_msg_body(eab_ref, xsb_ref,
                 w1_ref, b1_ref, w2_ref, b2_ref, exp_ref, redp_ref, out_ref):
    eab = eab_ref[...]
    xsb = xsb_ref[...]
    acc = jnp.zeros((BE // 8, 128), jnp.float32)
    for a in range(8):
        t = jnp.maximum(
            jnp.dot(eab[:, 16 * a:16 * (a + 1)].astype(jnp.bfloat16),
                    w1_ref[...].astype(jnp.bfloat16),
                    preferred_element_type=jnp.float32) + b1_ref[...], 0.0)
        h = jnp.dot(t.astype(jnp.bfloat16),
                    w2_ref[...].astype(jnp.bfloat16),
                    preferred_element_type=jnp.float32) + b2_ref[...]
        xs_rep = jnp.dot(xsb[:, 16 * a:16 * (a + 1)], exp_ref[...],
                         preferred_element_type=jnp.float32)
        acc = acc + jnp.dot(xs_rep * h, redp_ref[a],
                            preferred_element_type=jnp.float32)
    out_ref[...] = acc


GEB = E // 8 // (BE // 8)

_tc_msg = pl.pallas_call(
    _tc_msg_body,
    grid=(GEB,),
    in_specs=[
        pl.BlockSpec((BE // 8, 128), lambda i: (i, 0)),
        pl.BlockSpec((BE // 8, 128), lambda i: (i, 0)),
        pl.BlockSpec((C, H), lambda i: (0, 0)),
        pl.BlockSpec((1, H), lambda i: (0, 0)),
        pl.BlockSpec((H, H), lambda i: (0, 0)),
        pl.BlockSpec((1, H), lambda i: (0, 0)),
        pl.BlockSpec((C, H), lambda i: (0, 0)),
        pl.BlockSpec((8, H, 128), lambda i: (0, 0, 0)),
    ],
    out_specs=pl.BlockSpec((BE // 8, 128), lambda i: (i, 0)),
    out_shape=jax.ShapeDtypeStruct((E // 8, 128), jnp.float32),
)


def _tc_combine1_body(x_ref, agg_ref, cnt_ref, root_ref, bias_ref, out_ref):
    aggv = agg_ref[0] + agg_ref[1]
    cntv = jnp.maximum(cnt_ref[0] + cnt_ref[1], 1.0)
    mean = aggv / cntv
    out_ref[...] = jnp.maximum(
        jnp.dot(x_ref[...], root_ref[...], preferred_element_type=jnp.float32)
        + mean + bias_ref[...], 0.0)


_tc_combine1 = pl.pallas_call(
    _tc_combine1_body,
    out_shape=jax.ShapeDtypeStruct((N, C), jnp.float32),
)


def _tc_combine2_body(h_ref, agg_ref, cnt_ref, root_ref, bias_ref, lw_ref,
                      lb_ref, out_ref):
    aggv = agg_ref[0] + agg_ref[1]
    cntv = jnp.maximum(cnt_ref[0] + cnt_ref[1], 1.0)
    h2 = jnp.maximum(
        jnp.dot(h_ref[...], root_ref[...], preferred_element_type=jnp.float32)
        + aggv / cntv + bias_ref[...], 0.0)
    out_ref[...] = jnp.dot(
        h2, lw_ref[...], preferred_element_type=jnp.float32) + lb_ref[...]


_tc_combine2 = pl.pallas_call(
    _tc_combine2_body,
    out_shape=jax.ShapeDtypeStruct((N, NCLS), jnp.float32),
)


def kernel(x, edge_index, edge_attr,
           en1_w1, en1_b1, en1_w2, en1_b2, root1, bias1,
           en2_w1, en2_b1, en2_w2, en2_b2, root2, bias2,
           lin_w, lin_b):
    src_sc = edge_index[0].reshape(8, E // 8).T.reshape(R, W)
    dst_sc = edge_index[1].reshape(8, E // 8).T.reshape(R, W)
    zeros = jnp.zeros((NPC, C), jnp.float32)
    ones = jnp.ones((W, C), jnp.float32)
    jidx = jnp.arange(H) // C
    exp_m = (jidx[None, :] == jnp.arange(C)[:, None]).astype(jnp.float32)
    la = jnp.arange(128)
    redp = ((la[None, None, :] // C == jnp.arange(8)[:, None, None])
            & (jnp.arange(H)[None, :, None] % C == la[None, None, :] % C)
            ).astype(jnp.float32)
    _sc_gather, _sc_scatter_cnt, _sc_scatter = _sc_kernels()

    eab = (edge_attr.T.reshape(C, 8, E // 8).transpose(2, 1, 0)
           .reshape(E // 8, 128))

    xs = _sc_gather(x, src_sc)
    msg1 = _tc_msg(eab,
                   xs.reshape(E // 8, 128), en1_w1, en1_b1.reshape(1, H),
                   en1_w2, en1_b2.reshape(1, H), exp_m, redp)
    agg1, cnt = _sc_scatter_cnt(msg1.reshape(E, C), dst_sc, zeros, ones)
    h1 = _tc_combine1(x, agg1, cnt, root1, bias1.reshape(1, C))

    hs = _sc_gather(h1, src_sc)
    msg2 = _tc_msg(eab,
                   hs.reshape(E // 8, 128), en2_w1, en2_b1.reshape(1, H),
                   en2_w2, en2_b2.reshape(1, H), exp_m, redp)
    agg2 = _sc_scatter(msg2.reshape(E, C), dst_sc, zeros)
    return _tc_combine2(h1, agg2, cnt, root2, bias2.reshape(1, C),
                        lin_w, lin_b.reshape(1, NCLS))


# speedup vs baseline: 7.2077x; 1.0399x over previous
import jax
import jax.numpy as jnp
from jax import lax
from jax.experimental import pallas as pl
from jax.experimental.pallas import tpu as pltpu
from jax.experimental.pallas import tpu_sc as plsc

N = 10000
E = 320000
C = 16
H = 256
NCLS = 64
W = 128
R = E // W
NW = 32
KF = R // NW
TAIL = R - KF * NW
KB = 26
NB = KF // KB
SKB = 13
SNB = KF // SKB
NPC = N // 16


def _sc_gather_body(x_hbm, src_hbm, out_hbm, idx_v, rows_v, semi, semg,
                    semw):
    cid = lax.axis_index("c")
    sid = lax.axis_index("s")
    wid = sid * 2 + cid
    start = wid * KF

    @pl.when(wid < TAIL)
    def _():
        row = NW * KF + wid
        pltpu.sync_copy(src_hbm.at[row], idx_v.at[0, 0])
        pltpu.async_copy(x_hbm.at[idx_v.at[0, 0]],
                         rows_v.at[0, pl.ds(0, W)], semg).wait()
        pltpu.sync_copy(rows_v.at[0, pl.ds(0, W)],
                        out_hbm.at[pl.ds(row * W, W)])

    didx = {}
    dw = {}
    didx[0] = pltpu.async_copy(src_hbm.at[pl.ds(start, KB)], idx_v.at[0],
                               semi)
    for b in range(NB):
        p = b % 2
        didx[b].wait()
        if b >= 2:
            dw[b - 2].wait()
        gd = []
        for j in range(KB):
            gd.append(pltpu.async_copy(
                x_hbm.at[idx_v.at[p, j]], rows_v.at[p, pl.ds(j * W, W)],
                semg))
        if b + 1 < NB:
            didx[b + 1] = pltpu.async_copy(
                src_hbm.at[pl.ds(start + (b + 1) * KB, KB)],
                idx_v.at[(b + 1) % 2], semi)
        for g in gd:
            g.wait()
        dw[b] = pltpu.async_copy(
            rows_v.at[p], out_hbm.at[pl.ds((start + b * KB) * W, KB * W)],
            semw)
    for b in range(max(0, NB - 2), NB):
        dw[b].wait()


def _scatter_common(msg_hbm, dst_hbm, zeros_hbm, agg_out, idx_v, rows_v,
                    stage_v, agg_s, semi, semr, sema, extra_row=None,
                    extra_out=None, ones_v=None, cnt_s=None):
    cid = lax.axis_index("c")
    sid = lax.axis_index("s")
    wid = sid * 2 + cid
    start = wid * KF

    pltpu.sync_copy(zeros_hbm, stage_v)
    pltpu.sync_copy(stage_v, agg_s.at[pl.ds(sid * NPC, NPC)])
    if cnt_s is not None:
        pltpu.sync_copy(stage_v, cnt_s.at[pl.ds(sid * NPC, NPC)])
        pltpu.sync_copy(extra_row, ones_v)
    plsc.subcore_barrier()

    didx = {}
    dmsg = {}
    adds = {0: [], 1: []}
    didx[0] = pltpu.async_copy(dst_hbm.at[pl.ds(start, SKB)], idx_v.at[0],
                               semi)
    dmsg[0] = pltpu.async_copy(msg_hbm.at[pl.ds(start * W, SKB * W)],
                               rows_v.at[0], semr)
    for b in range(SNB):
        p = b % 2
        didx[b].wait()
        dmsg[b].wait()
        if b >= 2:
            for a in adds[p]:
                a.wait()
        adds[p] = []
        for j in range(SKB):
            adds[p].append(pltpu.async_copy(
                rows_v.at[p, pl.ds(j * W, W)], agg_s.at[idx_v.at[p, j]],
                sema, add=True))
            if cnt_s is not None:
                adds[p].append(pltpu.async_copy(
                    ones_v, cnt_s.at[idx_v.at[p, j]], sema, add=True))
        if b + 1 < SNB:
            row0 = start + (b + 1) * SKB
            didx[b + 1] = pltpu.async_copy(
                dst_hbm.at[pl.ds(row0, SKB)], idx_v.at[(b + 1) % 2], semi)
            dmsg[b + 1] = pltpu.async_copy(
                msg_hbm.at[pl.ds(row0 * W, SKB * W)],
                rows_v.at[(b + 1) % 2], semr)
    for p in (0, 1):
        for a in adds[p]:
            a.wait()

    @pl.when(wid < TAIL)
    def _():
        row = NW * KF + wid
        pltpu.sync_copy(dst_hbm.at[row], idx_v.at[0, 0])
        pltpu.sync_copy(msg_hbm.at[pl.ds(row * W, W)],
                        rows_v.at[0, pl.ds(0, W)])
        pltpu.sync_copy(rows_v.at[0, pl.ds(0, W)], agg_s.at[idx_v.at[0, 0]],
                        add=True)
        if cnt_s is not None:
            pltpu.sync_copy(ones_v, cnt_s.at[idx_v.at[0, 0]], add=True)

    plsc.subcore_barrier()

    def drain(src_s, dst_hbm_out):
        pltpu.sync_copy(src_s.at[pl.ds(sid * NPC, NPC)], stage_v)
        pltpu.sync_copy(stage_v, dst_hbm_out.at[cid, pl.ds(sid * NPC, NPC)])

    drain(agg_s, agg_out)
    if cnt_s is not None:
        drain(cnt_s, extra_out)


def _sc_scatter_cnt_body(msg_hbm, dst_hbm, zeros_hbm, ones_hbm, agg_out,
                         cnt_out, idx_v, rows_v, stage_v, ones_v, agg_s,
                         cnt_s, semi, semr, sema):
    _scatter_common(msg_hbm, dst_hbm, zeros_hbm, agg_out, idx_v, rows_v,
                    stage_v, agg_s, semi, semr, sema, extra_row=ones_hbm,
                    extra_out=cnt_out, ones_v=ones_v, cnt_s=cnt_s)


def _sc_scatter_body(msg_hbm, dst_hbm, zeros_hbm, agg_out, idx_v, rows_v,
                     stage_v, agg_s, semi, semr, sema):
    _scatter_common(msg_hbm, dst_hbm, zeros_hbm, agg_out, idx_v, rows_v,
                    stage_v, agg_s, semi, semr, sema)


import functools


@functools.lru_cache(maxsize=1)
def _sc_kernels():
    mesh = plsc.VectorSubcoreMesh(core_axis_name="c", subcore_axis_name="s")
    params = pltpu.CompilerParams(use_tc_tiling_on_sc=False)
    gather = pl.kernel(
        _sc_gather_body,
        out_type=jax.ShapeDtypeStruct((E, C), jnp.float32),
        mesh=mesh,
        compiler_params=params,
        scratch_types=[
            pltpu.VMEM((2, KB, W), jnp.int32),
            pltpu.VMEM((2, KB * W, C), jnp.float32),
            pltpu.SemaphoreType.DMA,
            pltpu.SemaphoreType.DMA,
            pltpu.SemaphoreType.DMA,
        ],
    )
    scatter_cnt = pl.kernel(
        _sc_scatter_cnt_body,
        compiler_params=params,
        out_type=(
            jax.ShapeDtypeStruct((2, N, C), jnp.float32),
            jax.ShapeDtypeStruct((2, N, C), jnp.float32),
        ),
        mesh=mesh,
        scratch_types=[
            pltpu.VMEM((2, SKB, W), jnp.int32),
            pltpu.VMEM((2, SKB * W, C), jnp.float32),
            pltpu.VMEM((NPC, C), jnp.float32),
            pltpu.VMEM((W, C), jnp.float32),
            pltpu.VMEM_SHARED((N, C), jnp.float32),
            pltpu.VMEM_SHARED((N, C), jnp.float32),
            pltpu.SemaphoreType.DMA,
            pltpu.SemaphoreType.DMA,
            pltpu.SemaphoreType.DMA,
        ],
    )
    scatter = pl.kernel(
        _sc_scatter_body,
        compiler_params=params,
        out_type=jax.ShapeDtypeStruct((2, N, C), jnp.float32),
        mesh=mesh,
        scratch_types=[
            pltpu.VMEM((2, SKB, W), jnp.int32),
            pltpu.VMEM((2, SKB * W, C), jnp.float32),
            pltpu.VMEM((NPC, C), jnp.float32),
            pltpu.VMEM_SHARED((N, C), jnp.float32),
            pltpu.SemaphoreType.DMA,
            pltpu.SemaphoreType.DMA,
            pltpu.SemaphoreType.DMA,
        ],
    )
    return gather, scatter_cnt, scatter


BE = 12800
GE = E // BE


def _tc_msg_body(eab_ref, xsb_ref,
                 w1_ref, b1_ref, w2_ref, b2_ref, exp_ref, redp_ref, out_ref):
    eab = eab_ref[...]
    xsb = xsb_ref[...]
    acc = jnp.zeros((BE // 8, 128), jnp.float32)
    for a in range(8):
        t = jnp.maximum(
            jnp.dot(eab[:, 16 * a:16 * (a + 1)].astype(jnp.bfloat16),
                    w1_ref[...].astype(jnp.bfloat16),
                    preferred_element_type=jnp.float32) + b1_ref[...], 0.0)
        h = jnp.dot(t.astype(jnp.bfloat16),
                    w2_ref[...].astype(jnp.bfloat16),
                    preferred_element_type=jnp.float32) + b2_ref[...]
        xs_rep = jnp.dot(xsb[:, 16 * a:16 * (a + 1)], exp_ref[...],
                         preferred_element_type=jnp.float32)
        acc = acc + jnp.dot(xs_rep * h, redp_ref[a],
                            preferred_element_type=jnp.float32)
    out_ref[...] = acc


GEB = E // 8 // (BE // 8)

_tc_msg = pl.pallas_call(
    _tc_msg_body,
    grid=(GEB,),
    in_specs=[
        pl.BlockSpec((BE // 8, 128), lambda i: (i, 0)),
        pl.BlockSpec((BE // 8, 128), lambda i: (i, 0)),
        pl.BlockSpec((C, H), lambda i: (0, 0)),
        pl.BlockSpec((1, H), lambda i: (0, 0)),
        pl.BlockSpec((H, H), lambda i: (0, 0)),
        pl.BlockSpec((1, H), lambda i: (0, 0)),
        pl.BlockSpec((C, H), lambda i: (0, 0)),
        pl.BlockSpec((8, H, 128), lambda i: (0, 0, 0)),
    ],
    out_specs=pl.BlockSpec((BE // 8, 128), lambda i: (i, 0)),
    out_shape=jax.ShapeDtypeStruct((E // 8, 128), jnp.float32),
)


def _tc_combine1_body(x_ref, agg_ref, cnt_ref, root_ref, bias_ref, out_ref):
    aggv = agg_ref[0] + agg_ref[1]
    cntv = jnp.maximum(cnt_ref[0] + cnt_ref[1], 1.0)
    mean = aggv / cntv
    out_ref[...] = jnp.maximum(
        jnp.dot(x_ref[...], root_ref[...], preferred_element_type=jnp.float32)
        + mean + bias_ref[...], 0.0)


_tc_combine1 = pl.pallas_call(
    _tc_combine1_body,
    out_shape=jax.ShapeDtypeStruct((N, C), jnp.float32),
)


def _tc_combine2_body(h_ref, agg_ref, cnt_ref, root_ref, bias_ref, lw_ref,
                      lb_ref, out_ref):
    aggv = agg_ref[0] + agg_ref[1]
    cntv = jnp.maximum(cnt_ref[0] + cnt_ref[1], 1.0)
    h2 = jnp.maximum(
        jnp.dot(h_ref[...], root_ref[...], preferred_element_type=jnp.float32)
        + aggv / cntv + bias_ref[...], 0.0)
    out_ref[...] = jnp.dot(
        h2, lw_ref[...], preferred_element_type=jnp.float32) + lb_ref[...]


_tc_combine2 = pl.pallas_call(
    _tc_combine2_body,
    out_shape=jax.ShapeDtypeStruct((N, NCLS), jnp.float32),
)


def kernel(x, edge_index, edge_attr,
           en1_w1, en1_b1, en1_w2, en1_b2, root1, bias1,
           en2_w1, en2_b1, en2_w2, en2_b2, root2, bias2,
           lin_w, lin_b):
    src_sc = edge_index[0].reshape(8, E // 8).T.reshape(R, W)
    dst_sc = edge_index[1].reshape(8, E // 8).T.reshape(R, W)
    zeros = jnp.zeros((NPC, C), jnp.float32)
    ones = jnp.ones((W, C), jnp.float32)
    jidx = jnp.arange(H) // C
    exp_m = (jidx[None, :] == jnp.arange(C)[:, None]).astype(jnp.float32)
    la = jnp.arange(128)
    redp = ((la[None, None, :] // C == jnp.arange(8)[:, None, None])
            & (jnp.arange(H)[None, :, None] % C == la[None, None, :] % C)
            ).astype(jnp.float32)
    _sc_gather, _sc_scatter_cnt, _sc_scatter = _sc_kernels()

    eab = (edge_attr.T.reshape(C, 8, E // 8).transpose(2, 1, 0)
           .reshape(E // 8, 128))

    xs = _sc_gather(x, src_sc)
    msg1 = _tc_msg(eab,
                   xs.reshape(E // 8, 128), en1_w1, en1_b1.reshape(1, H),
                   en1_w2, en1_b2.reshape(1, H), exp_m, redp)
    agg1, cnt = _sc_scatter_cnt(msg1.reshape(E, C), dst_sc, zeros, ones)
    h1 = _tc_combine1(x, agg1, cnt, root1, bias1.reshape(1, C))

    hs = _sc_gather(h1, src_sc)
    msg2 = _tc_msg(eab,
                   hs.reshape(E // 8, 128), en2_w1, en2_b1.reshape(1, H),
                   en2_w2, en2_b2.reshape(1, H), exp_m, redp)
    agg2 = _sc_scatter(msg2.reshape(E, C), dst_sc, zeros)
    return _tc_combine2(h1, agg2, cnt, root2, bias2.reshape(1, C),
                        lin_w, lin_b.reshape(1, NCLS))
